# Initial kernel scaffold; baseline (speedup 1.0000x reference)
import functools
import numpy as np
import jax
import jax.numpy as jnp
from jax import lax
from jax.experimental import pallas as pl
from jax.experimental.pallas import tpu as pltpu
from jax.experimental.pallas import tpu_sc as plsc

NC = 2
NS = 16
NWORK = NC * NS

F32 = jnp.float32


def _dense1_body(x_ref, w1_ref, a1s_ref, a1d_ref, h_ref, as_ref, ad_ref):
    h = jnp.dot(x_ref[...], w1_ref[...], preferred_element_type=F32)
    h_ref[...] = h
    as_ref[...] = jnp.dot(h, a1s_ref[...], preferred_element_type=F32)
    ad_ref[...] = jnp.dot(h, a1d_ref[...], preferred_element_type=F32)


def _dense1(x, W1, A1s, A1d, bn):
    n, din = x.shape
    dh = W1.shape[1]
    grid = (n // bn,)
    return pl.pallas_call(
        _dense1_body,
        grid=grid,
        in_specs=[
            pl.BlockSpec((bn, din), lambda i: (i, 0)),
            pl.BlockSpec((din, dh), lambda i: (0, 0)),
            pl.BlockSpec((dh, 16), lambda i: (0, 0)),
            pl.BlockSpec((dh, 16), lambda i: (0, 0)),
        ],
        out_specs=[
            pl.BlockSpec((bn, dh), lambda i: (i, 0)),
            pl.BlockSpec((bn, 16), lambda i: (i, 0)),
            pl.BlockSpec((bn, 16), lambda i: (i, 0)),
        ],
        out_shape=[
            jax.ShapeDtypeStruct((n, dh), F32),
            jax.ShapeDtypeStruct((n, 16), F32),
            jax.ShapeDtypeStruct((n, 16), F32),
        ],
    )(x, W1, A1s, A1d)


def _make_edge1(n, e, k1):
    epw = e // NWORK
    nwin = epw // k1
    rows_pt = n // NS
    mesh = plsc.VectorSubcoreMesh(core_axis_name="c", subcore_axis_name="s")

    @functools.partial(
        pl.kernel,
        mesh=mesh,
        out_type=(
            jax.ShapeDtypeStruct((NC, n, 128), F32),
            jax.ShapeDtypeStruct((NC, n, 16), F32),
        ),
        scratch_types=[
            pltpu.VMEM_SHARED((n, 128), F32),
            pltpu.VMEM_SHARED((n, 16), F32),
            pltpu.VMEM((k1,), jnp.int32),
            pltpu.VMEM((k1,), jnp.int32),
            pltpu.VMEM((k1, 16), F32),
            pltpu.VMEM((k1, 16), F32),
            pltpu.VMEM((k1, 16), F32),
            pltpu.VMEM((k1, 128), F32),
            pltpu.VMEM((k1, 128), F32),
            pltpu.SemaphoreType.DMA,
        ],
    )
    def edge1(src_hbm, dst_hbm, h_hbm, as_hbm, ad_hbm, z128_hbm, z16_hbm,
              acc_out, den_out,
              acc_sp, den_sp, sidx, didx, asr, adr, exr, hrows, msg, sem):
        cid = lax.axis_index("c")
        sid = lax.axis_index("s")
        wid = sid * NC + cid
        r0 = sid * rows_pt
        pltpu.sync_copy(z128_hbm.at[pl.ds(r0, rows_pt)],
                        acc_sp.at[pl.ds(r0, rows_pt)])
        pltpu.sync_copy(z16_hbm.at[pl.ds(r0, rows_pt)],
                        den_sp.at[pl.ds(r0, rows_pt)])
        plsc.subcore_barrier()

        base = wid * epw

        def window(w, carry):
            off = base + w * k1
            pltpu.sync_copy(src_hbm.at[pl.ds(off, k1)], sidx)
            pltpu.sync_copy(dst_hbm.at[pl.ds(off, k1)], didx)
            pltpu.async_copy(as_hbm.at[sidx], asr, sem).wait()
            pltpu.async_copy(ad_hbm.at[didx], adr, sem).wait()
            pltpu.async_copy(h_hbm.at[sidx], hrows, sem).wait()

            def per_edge(k, c):
                v = asr[k, :] + adr[k, :]
                v = jnp.where(v > 0, v, v * 0.2)
                ex = jnp.exp(v)
                exr[k, :] = ex
                for hh in range(8):
                    s = ex[hh]
                    msg[k, pl.ds(hh * 16, 16)] = (
                        hrows[k, pl.ds(hh * 16, 16)] * s)
                return c

            lax.fori_loop(0, k1, per_edge, 0)
            pltpu.sync_copy(exr, den_sp.at[didx], add=True)
            pltpu.sync_copy(msg, acc_sp.at[didx], add=True)
            return carry

        lax.fori_loop(0, nwin, window, 0)
        plsc.subcore_barrier()
        pltpu.sync_copy(acc_sp.at[pl.ds(r0, rows_pt)],
                        acc_out.at[cid, pl.ds(r0, rows_pt)])
        pltpu.sync_copy(den_sp.at[pl.ds(r0, rows_pt)],
                        den_out.at[cid, pl.ds(r0, rows_pt)])

    return edge1


def _mid_body(acc_ref, den_ref, rexp_ref, b1_ref, g_ref, bt_ref, mu_ref,
              va_ref, w2_ref, a2s_ref, a2d_ref, h2_ref, as2_ref, ad2_ref):
    a = acc_ref[0] + acc_ref[1]
    d = den_ref[0] + den_ref[1]
    dexp = jnp.dot(d, rexp_ref[...], preferred_element_type=F32)
    h1 = a / (dexp + 1e-16) + b1_ref[...]
    scale = g_ref[...] * lax.rsqrt(va_ref[...] + 1e-5)
    h1 = (h1 - mu_ref[...]) * scale + bt_ref[...]
    h1 = jnp.where(h1 > 0, h1, jnp.expm1(h1))
    h2 = jnp.dot(h1, w2_ref[...], preferred_element_type=F32)
    h2_ref[...] = h2
    as2_ref[...] = jnp.dot(h2, a2s_ref[...], preferred_element_type=F32)
    ad2_ref[...] = jnp.dot(h2, a2d_ref[...], preferred_element_type=F32)


def _mid(acc1, den1, Rexp, b1r, g, bt, mu, va, W2, a2s, a2d, bn):
    n = acc1.shape[1]
    hid = W2.shape[1]
    grid = (n // bn,)
    return pl.pallas_call(
        _mid_body,
        grid=grid,
        in_specs=[
            pl.BlockSpec((2, bn, 128), lambda i: (0, i, 0)),
            pl.BlockSpec((2, bn, 16), lambda i: (0, i, 0)),
            pl.BlockSpec((16, 128), lambda i: (0, 0)),
            pl.BlockSpec((1, 128), lambda i: (0, 0)),
            pl.BlockSpec((1, 128), lambda i: (0, 0)),
            pl.BlockSpec((1, 128), lambda i: (0, 0)),
            pl.BlockSpec((1, 128), lambda i: (0, 0)),
            pl.BlockSpec((1, 128), lambda i: (0, 0)),
            pl.BlockSpec((128, hid), lambda i: (0, 0)),
            pl.BlockSpec((hid, 128), lambda i: (0, 0)),
            pl.BlockSpec((hid, 128), lambda i: (0, 0)),
        ],
        out_specs=[
            pl.BlockSpec((bn, hid), lambda i: (i, 0)),
            pl.BlockSpec((bn, 128), lambda i: (i, 0)),
            pl.BlockSpec((bn, 128), lambda i: (i, 0)),
        ],
        out_shape=[
            jax.ShapeDtypeStruct((n, hid), F32),
            jax.ShapeDtypeStruct((n, 128), F32),
            jax.ShapeDtypeStruct((n, 128), F32),
        ],
    )(acc1, den1, Rexp, b1r, g, bt, mu, va, W2, a2s, a2d)


def _make_edge2(n, e, k2):
    epw = e // NWORK
    nwin = epw // k2
    rows_pt = n // NS
    ngrp = k2 // 16
    mesh = plsc.VectorSubcoreMesh(core_axis_name="c", subcore_axis_name="s")

    @functools.partial(
        pl.kernel,
        mesh=mesh,
        out_type=jax.ShapeDtypeStruct((NC, n, 32), F32),
        scratch_types=[
            pltpu.VMEM_SHARED((n, 32), F32),
            pltpu.VMEM((n,), F32),
            pltpu.VMEM((n,), F32),
            pltpu.VMEM((k2,), jnp.int32),
            pltpu.VMEM((k2,), jnp.int32),
            pltpu.VMEM((k2, 16), F32),
            pltpu.VMEM((k2, 32), F32),
            pltpu.SemaphoreType.DMA,
        ],
    )
    def edge2(src_hbm, dst_hbm, h2_hbm, as_hbm, ad_hbm, z32_hbm,
              acc_out,
              acc_sp, asv, adv, sidx, didx, h2r, msg, sem):
        cid = lax.axis_index("c")
        sid = lax.axis_index("s")
        wid = sid * NC + cid
        r0 = sid * rows_pt
        pltpu.sync_copy(z32_hbm.at[pl.ds(r0, rows_pt)],
                        acc_sp.at[pl.ds(r0, rows_pt)])
        pltpu.sync_copy(as_hbm, asv)
        pltpu.sync_copy(ad_hbm, adv)
        plsc.subcore_barrier()

        lane = lax.iota(jnp.int32, 16)
        base = wid * epw

        def window(w, carry):
            off = base + w * k2
            pltpu.sync_copy(src_hbm.at[pl.ds(off, k2)], sidx)
            pltpu.sync_copy(dst_hbm.at[pl.ds(off, k2)], didx)
            pltpu.async_copy(h2_hbm.at[sidx], h2r, sem).wait()

            def per_group(j, c):
                sv = plsc.load_gather(asv, [sidx[pl.ds(j * 16, 16)]])
                dv = plsc.load_gather(adv, [didx[pl.ds(j * 16, 16)]])
                v = sv + dv
                v = jnp.where(v > 0, v, v * 0.2)
                ex = jnp.exp(v)
                for t in range(16):
                    k = j * 16 + t
                    s = ex[t]
                    msg[k, pl.ds(0, 16)] = h2r[k, :] * s
                    msg[k, pl.ds(16, 16)] = jnp.where(lane == 0, s, 0.0)
                return c

            lax.fori_loop(0, ngrp, per_group, 0)
            pltpu.sync_copy(msg, acc_sp.at[didx], add=True)
            return carry

        lax.fori_loop(0, nwin, window, 0)
        plsc.subcore_barrier()
        pltpu.sync_copy(acc_sp.at[pl.ds(r0, rows_pt)],
                        acc_out.at[cid, pl.ds(r0, rows_pt)])

    return edge2


def _head_body(acc_ref, nsel_ref, dsel_ref, b2_ref, wc_ref, bc_ref, ws_ref,
               bs_ref, wh_ref, bh_ref, we_ref, be_ref,
               main_ref, sim_ref, hom_ref, ent_ref):
    s = acc_ref[0] + acc_ref[1]
    num = jnp.dot(s, nsel_ref[...], preferred_element_type=F32)
    den = jnp.dot(s, dsel_ref[...], preferred_element_type=F32)
    h = num / (den + 1e-16) + b2_ref[...]
    mo = jnp.dot(h, wc_ref[...], preferred_element_type=F32) + bc_ref[...]
    mo = mo - jnp.max(mo, axis=1, keepdims=True)
    main_ref[...] = mo - jnp.log(jnp.sum(jnp.exp(mo), axis=1, keepdims=True))
    so = jnp.dot(h, ws_ref[...], preferred_element_type=F32) + bs_ref[...]
    so = jnp.exp(so - jnp.max(so, axis=1, keepdims=True))
    sim_ref[...] = so / jnp.sum(so, axis=1, keepdims=True)
    hom_ref[...] = jax.nn.sigmoid(
        jnp.dot(h, wh_ref[...], preferred_element_type=F32) + bh_ref[...])
    ent_ref[...] = jax.nn.sigmoid(
        jnp.dot(h, we_ref[...], preferred_element_type=F32) + be_ref[...])


def _heads(acc2, Nsel, Dsel, b2r, Wc, bcr, Ws, bsr, Wh, bhr, We, ber, bn):
    n = acc2.shape[1]
    out_c = Wc.shape[1]
    ncls = Ws.shape[1]
    grid = (n // bn,)

    def cst(shp):
        return pl.BlockSpec(shp, lambda *_: tuple(0 for _ in shp))

    return pl.pallas_call(
        _head_body,
        grid=grid,
        in_specs=[
            pl.BlockSpec((2, bn, 32), lambda i: (0, i, 0)),
            cst((32, 16)), cst((32, 16)), cst((1, 16)),
            cst((16, out_c)), cst((1, out_c)),
            cst((16, ncls)), cst((1, ncls)),
            cst((16, 1)), cst((1, 1)),
            cst((16, 1)), cst((1, 1)),
        ],
        out_specs=[
            pl.BlockSpec((bn, out_c), lambda i: (i, 0)),
            pl.BlockSpec((bn, ncls), lambda i: (i, 0)),
            pl.BlockSpec((bn, 1), lambda i: (i, 0)),
            pl.BlockSpec((bn, 1), lambda i: (i, 0)),
        ],
        out_shape=[
            jax.ShapeDtypeStruct((n, out_c), F32),
            jax.ShapeDtypeStruct((n, ncls), F32),
            jax.ShapeDtypeStruct((n, 1), F32),
            jax.ShapeDtypeStruct((n, 1), F32),
        ],
    )(acc2, Nsel, Dsel, b2r, Wc, bcr, Ws, bsr, Wh, bhr, We, ber)


def kernel(x, edge_index, W1, att_src1, att_dst1, b1, bn_gamma, bn_beta,
           bn_mean, bn_var, W2, att_src2, att_dst2, b2, Wc, bc, Ws, bs,
           Wh, bh, We, be):
    n, din = x.shape
    e = edge_index.shape[1]
    heads, hid = att_src1.shape
    dh = heads * hid

    src = edge_index[0]
    dst = edge_index[1]

    hsel = np.zeros((heads, dh, 16), np.float32)
    for h in range(heads):
        hsel[h, h * hid:(h + 1) * hid, h] = 1.0
    hsel = jnp.asarray(hsel)
    A1s = jnp.einsum("hc,hco->co", att_src1,
                     hsel.reshape(heads, dh, 16)[:, :, :]
                     .reshape(heads, heads, hid, 16)[
                         jnp.arange(heads), jnp.arange(heads)])
    A1d = jnp.einsum("hc,hco->co", att_dst1,
                     hsel.reshape(heads, heads, hid, 16)[
                         jnp.arange(heads), jnp.arange(heads)])

    rexp = np.zeros((16, dh), np.float32)
    for h in range(heads):
        rexp[h, h * hid:(h + 1) * hid] = 1.0
    Rexp = jnp.asarray(rexp)

    nsel = np.zeros((32, 16), np.float32)
    nsel[:16, :16] = np.eye(16, dtype=np.float32)
    dsel = np.zeros((32, 16), np.float32)
    dsel[16, :] = 1.0
    Nsel = jnp.asarray(nsel)
    Dsel = jnp.asarray(dsel)

    bn = 1000
    h1, as1, ad1 = _dense1(x, W1, A1s, A1d, bn)

    z128 = jnp.zeros((n, 128), F32)
    z16 = jnp.zeros((n, 16), F32)
    z32 = jnp.zeros((n, 32), F32)

    edge1 = _make_edge1(n, e, 200)
    acc1, den1 = edge1(src, dst, h1, as1, ad1, z128, z16)

    h2, as2, ad2 = _mid(
        acc1, den1, Rexp, b1.reshape(1, dh), bn_gamma.reshape(1, dh),
        bn_beta.reshape(1, dh), bn_mean.reshape(1, dh), bn_var.reshape(1, dh),
        W2, jnp.tile(att_src2.reshape(hid, 1), (1, 128)),
        jnp.tile(att_dst2.reshape(hid, 1), (1, 128)), bn)

    edge2 = _make_edge2(n, e, 400)
    acc2 = edge2(src, dst, h2, as2[:, 0], ad2[:, 0], z32)

    main, sim, hom, ent = _heads(
        acc2, Nsel, Dsel, b2.reshape(1, hid), Wc, bc.reshape(1, -1),
        Ws, bs.reshape(1, -1), Wh, bh.reshape(1, 1), We, be.reshape(1, 1),
        bn)
    return main, sim, hom[:, 0], ent[:, 0]


# speedup vs baseline: 29.1752x; 29.1752x over previous
import functools
import numpy as np
import jax
import jax.numpy as jnp
from jax import lax
from jax.experimental import pallas as pl
from jax.experimental.pallas import tpu as pltpu
from jax.experimental.pallas import tpu_sc as plsc

NC = 2
NS = 16
NWORK = NC * NS
KW = 128

F32 = jnp.float32


def _dense1_body(x_ref, w1_ref, a1s_ref, a1d_ref, h_ref, as_ref, ad_ref):
    h = jnp.dot(x_ref[...], w1_ref[...], preferred_element_type=F32)
    h_ref[...] = h
    as_ref[...] = jnp.dot(h, a1s_ref[...], preferred_element_type=F32)
    ad_ref[...] = jnp.dot(h, a1d_ref[...], preferred_element_type=F32)


def _dense1(x, W1, A1s, A1d, bn):
    n, din = x.shape
    dh = W1.shape[1]
    grid = (n // bn,)
    return pl.pallas_call(
        _dense1_body,
        grid=grid,
        in_specs=[
            pl.BlockSpec((bn, din), lambda i: (i, 0)),
            pl.BlockSpec((din, dh), lambda i: (0, 0)),
            pl.BlockSpec((dh, 16), lambda i: (0, 0)),
            pl.BlockSpec((dh, 16), lambda i: (0, 0)),
        ],
        out_specs=[
            pl.BlockSpec((bn, dh), lambda i: (i, 0)),
            pl.BlockSpec((bn, 16), lambda i: (i, 0)),
            pl.BlockSpec((bn, 16), lambda i: (i, 0)),
        ],
        out_shape=[
            jax.ShapeDtypeStruct((n, dh), F32),
            jax.ShapeDtypeStruct((n, 16), F32),
            jax.ShapeDtypeStruct((n, 16), F32),
        ],
    )(x, W1, A1s, A1d)


def _make_edge1(npad, e):
    k1 = KW
    nwin_total = e // k1
    nbase = nwin_total // NWORK
    nrem = nwin_total % NWORK
    rows_pt = npad // NS
    nchunk = rows_pt // 128
    mesh = plsc.VectorSubcoreMesh(core_axis_name="c", subcore_axis_name="s")

    @functools.partial(
        pl.kernel,
        mesh=mesh,
        out_type=(
            jax.ShapeDtypeStruct((NC, npad, 128), F32),
            jax.ShapeDtypeStruct((NC, npad * 16), F32),
        ),
        scratch_types=[
            pltpu.VMEM_SHARED((npad, 128), F32),
            pltpu.VMEM_SHARED((npad * 16,), F32),
            pltpu.VMEM((k1,), jnp.int32),
            pltpu.VMEM((k1,), jnp.int32),
            pltpu.VMEM((k1 * 16,), jnp.int32),
            pltpu.VMEM((k1 * 16,), jnp.int32),
            pltpu.VMEM((k1 * 16,), F32),
            pltpu.VMEM((k1 * 16,), F32),
            pltpu.VMEM((k1, 128), F32),
            pltpu.VMEM((2048,), F32),
            pltpu.SemaphoreType.DMA,
        ],
    )
    def edge1(src_hbm, dst_hbm, h_hbm, asf_hbm, adf_hbm,
              acc_out, den_out,
              acc_sp, den_sp, sidx, didx, idx1, idx2, asr, adr,
              hrows, zbuf, sem):
        cid = lax.axis_index("c")
        sid = lax.axis_index("s")
        wid = sid * NC + cid
        r0 = sid * rows_pt
        lane = lax.iota(jnp.int32, 16)

        def zrow(r, c):
            for hh in range(8):
                hrows[r, pl.ds(hh * 16, 16)] = jnp.zeros((16,), F32)
            return c

        lax.fori_loop(0, k1, zrow, 0)

        def zb(i, c):
            zbuf[pl.ds(i * 16, 16)] = jnp.zeros((16,), F32)
            return c

        lax.fori_loop(0, 128, zb, 0)
        for c in range(nchunk):
            pltpu.sync_copy(hrows, acc_sp.at[pl.ds(r0 + c * 128, 128)])
            pltpu.sync_copy(zbuf, den_sp.at[pl.ds((r0 + c * 128) * 16, 2048)])
        plsc.subcore_barrier()

        nwin = nbase + jnp.where(wid < nrem, 1, 0)

        def window(w, carry):
            off = (wid + NWORK * w) * k1
            pltpu.sync_copy(src_hbm.at[pl.ds(off, k1)], sidx)
            pltpu.sync_copy(dst_hbm.at[pl.ds(off, k1)], didx)

            def idx_group(j, c):
                svec = sidx[pl.ds(j * 16, 16)]
                dvec = didx[pl.ds(j * 16, 16)]
                for t in range(16):
                    k = j * 16 + t
                    idx1[pl.ds(k * 16, 16)] = svec[t] * 16 + lane
                    idx2[pl.ds(k * 16, 16)] = dvec[t] * 16 + lane
                return c

            lax.fori_loop(0, k1 // 16, idx_group, 0)
            pltpu.async_copy(asf_hbm.at[idx1], asr, sem).wait()
            pltpu.async_copy(adf_hbm.at[idx2], adr, sem).wait()
            pltpu.async_copy(h_hbm.at[sidx], hrows, sem).wait()

            def per_edge(k, c):
                v = asr[pl.ds(k * 16, 16)] + adr[pl.ds(k * 16, 16)]
                v = jnp.where(v > 0, v, v * 0.2)
                ex = jnp.exp(v)
                asr[pl.ds(k * 16, 16)] = ex
                for hh in range(8):
                    hrows[k, pl.ds(hh * 16, 16)] = (
                        hrows[k, pl.ds(hh * 16, 16)] * ex[hh])
                return c

            lax.fori_loop(0, k1, per_edge, 0)
            pltpu.sync_copy(asr, den_sp.at[idx2], add=True)
            pltpu.sync_copy(hrows, acc_sp.at[didx], add=True)
            return carry

        lax.fori_loop(0, nwin, window, 0)
        plsc.subcore_barrier()

        for c in range(nchunk):
            r = r0 + c * 128
            pltpu.sync_copy(acc_sp.at[pl.ds(r, 128)], hrows)
            pltpu.sync_copy(hrows, acc_out.at[cid, pl.ds(r, 128)])
            pltpu.sync_copy(den_sp.at[pl.ds(r * 16, 2048)], zbuf)
            pltpu.sync_copy(zbuf, den_out.at[cid, pl.ds(r * 16, 2048)])

    return edge1


def _mid_body(acc_ref, den_ref, rexp_ref, b1_ref, g_ref, bt_ref, mu_ref,
              va_ref, w2_ref, a2s_ref, a2d_ref, h2_ref, as2_ref, ad2_ref):
    a = acc_ref[0] + acc_ref[1]
    d = den_ref[0] + den_ref[1]
    dexp = jnp.dot(d, rexp_ref[...], preferred_element_type=F32)
    h1 = a / (dexp + 1e-16) + b1_ref[...]
    scale = g_ref[...] * lax.rsqrt(va_ref[...] + 1e-5)
    h1 = (h1 - mu_ref[...]) * scale + bt_ref[...]
    h1 = jnp.where(h1 > 0, h1, jnp.exp(h1) - 1.0)
    h2 = jnp.dot(h1, w2_ref[...], preferred_element_type=F32)
    h2_ref[...] = h2
    as2_ref[...] = jnp.dot(h2, a2s_ref[...], preferred_element_type=F32)
    ad2_ref[...] = jnp.dot(h2, a2d_ref[...], preferred_element_type=F32)


def _mid(acc1, den1, Rexp, b1r, g, bt, mu, va, W2, a2s, a2d, n, bn):
    hid = W2.shape[1]
    grid = (n // bn,)
    return pl.pallas_call(
        _mid_body,
        grid=grid,
        in_specs=[
            pl.BlockSpec((2, bn, 128), lambda i: (0, i, 0)),
            pl.BlockSpec((2, bn, 16), lambda i: (0, i, 0)),
            pl.BlockSpec((16, 128), lambda i: (0, 0)),
            pl.BlockSpec((1, 128), lambda i: (0, 0)),
            pl.BlockSpec((1, 128), lambda i: (0, 0)),
            pl.BlockSpec((1, 128), lambda i: (0, 0)),
            pl.BlockSpec((1, 128), lambda i: (0, 0)),
            pl.BlockSpec((1, 128), lambda i: (0, 0)),
            pl.BlockSpec((128, hid), lambda i: (0, 0)),
            pl.BlockSpec((hid, 128), lambda i: (0, 0)),
            pl.BlockSpec((hid, 128), lambda i: (0, 0)),
        ],
        out_specs=[
            pl.BlockSpec((bn, hid), lambda i: (i, 0)),
            pl.BlockSpec((bn, 128), lambda i: (i, 0)),
            pl.BlockSpec((bn, 128), lambda i: (i, 0)),
        ],
        out_shape=[
            jax.ShapeDtypeStruct((n, hid), F32),
            jax.ShapeDtypeStruct((n, 128), F32),
            jax.ShapeDtypeStruct((n, 128), F32),
        ],
    )(acc1, den1, Rexp, b1r, g, bt, mu, va, W2, a2s, a2d)


def _make_edge2(npad, e):
    k2 = KW
    nwin_total = e // k2
    nbase = nwin_total // NWORK
    nrem = nwin_total % NWORK
    rows_pt = npad // NS
    nchunk = rows_pt // 128
    mesh = plsc.VectorSubcoreMesh(core_axis_name="c", subcore_axis_name="s")

    @functools.partial(
        pl.kernel,
        mesh=mesh,
        out_type=jax.ShapeDtypeStruct((NC, npad * 32), F32),
        scratch_types=[
            pltpu.VMEM_SHARED((npad * 32,), F32),
            pltpu.VMEM((k2,), F32),
            pltpu.VMEM((k2,), F32),
            pltpu.VMEM((k2,), jnp.int32),
            pltpu.VMEM((k2,), jnp.int32),
            pltpu.VMEM((k2 * 16,), jnp.int32),
            pltpu.VMEM((k2 * 32,), jnp.int32),
            pltpu.VMEM((k2 * 16,), F32),
            pltpu.VMEM((k2 * 32,), F32),
            pltpu.VMEM((4096,), F32),
            pltpu.SemaphoreType.DMA,
        ],
    )
    def edge2(src_hbm, dst_hbm, h2f_hbm, as_hbm, ad_hbm,
              acc_out,
              acc_sp, asr2, adr2, sidx, didx, idxg, idxm, h2r, msg,
              zbuf, sem):
        cid = lax.axis_index("c")
        sid = lax.axis_index("s")
        wid = sid * NC + cid
        r0 = sid * rows_pt
        lane = lax.iota(jnp.int32, 16)

        def zb(i, c):
            zbuf[pl.ds(i * 16, 16)] = jnp.zeros((16,), F32)
            return c

        lax.fori_loop(0, 256, zb, 0)
        for c in range(nchunk):
            pltpu.sync_copy(zbuf, acc_sp.at[pl.ds((r0 + c * 128) * 32, 4096)])
        plsc.subcore_barrier()

        nwin = nbase + jnp.where(wid < nrem, 1, 0)

        def window(w, carry):
            off = (wid + NWORK * w) * k2
            pltpu.sync_copy(src_hbm.at[pl.ds(off, k2)], sidx)
            pltpu.sync_copy(dst_hbm.at[pl.ds(off, k2)], didx)

            def idx_group(j, c):
                svec = sidx[pl.ds(j * 16, 16)]
                dvec = didx[pl.ds(j * 16, 16)]
                for t in range(16):
                    k = j * 16 + t
                    idxg[pl.ds(k * 16, 16)] = svec[t] * 16 + lane
                    idxm[pl.ds(k * 32, 16)] = dvec[t] * 32 + lane
                    idxm[pl.ds(k * 32 + 16, 16)] = dvec[t] * 32 + 16 + lane
                return c

            lax.fori_loop(0, k2 // 16, idx_group, 0)
            pltpu.async_copy(h2f_hbm.at[idxg], h2r, sem).wait()
            pltpu.async_copy(as_hbm.at[sidx], asr2, sem).wait()
            pltpu.async_copy(ad_hbm.at[didx], adr2, sem).wait()

            def per_group(j, c):
                sv = asr2[pl.ds(j * 16, 16)]
                dv = adr2[pl.ds(j * 16, 16)]
                v = sv + dv
                v = jnp.where(v > 0, v, v * 0.2)
                ex = jnp.exp(v)
                for t in range(16):
                    k = j * 16 + t
                    s = ex[t]
                    msg[pl.ds(k * 32, 16)] = h2r[pl.ds(k * 16, 16)] * s
                    msg[pl.ds(k * 32 + 16, 16)] = jnp.where(lane == 0, s, 0.0)
                return c

            lax.fori_loop(0, k2 // 16, per_group, 0)
            pltpu.sync_copy(msg, acc_sp.at[idxm], add=True)
            return carry

        lax.fori_loop(0, nwin, window, 0)
        plsc.subcore_barrier()
        for c in range(nchunk):
            f0 = (r0 + c * 128) * 32
            pltpu.sync_copy(acc_sp.at[pl.ds(f0, 4096)], zbuf)
            pltpu.sync_copy(zbuf, acc_out.at[cid, pl.ds(f0, 4096)])

    return edge2


def _head_body(acc_ref, nsel_ref, dsel_ref, b2_ref, wc_ref, bc_ref, ws_ref,
               bs_ref, wh_ref, bh_ref, we_ref, be_ref,
               main_ref, sim_ref, hom_ref, ent_ref):
    s = acc_ref[0] + acc_ref[1]
    num = jnp.dot(s, nsel_ref[...], preferred_element_type=F32)
    den = jnp.dot(s, dsel_ref[...], preferred_element_type=F32)
    h = num / (den + 1e-16) + b2_ref[...]
    mo = jnp.dot(h, wc_ref[...], preferred_element_type=F32) + bc_ref[...]
    mo = mo - jnp.max(mo, axis=1, keepdims=True)
    main_ref[...] = mo - jnp.log(jnp.sum(jnp.exp(mo), axis=1, keepdims=True))
    so = jnp.dot(h, ws_ref[...], preferred_element_type=F32) + bs_ref[...]
    so = jnp.exp(so - jnp.max(so, axis=1, keepdims=True))
    sim_ref[...] = so / jnp.sum(so, axis=1, keepdims=True)
    hom_ref[...] = jax.nn.sigmoid(
        jnp.dot(h, wh_ref[...], preferred_element_type=F32) + bh_ref[...])
    ent_ref[...] = jax.nn.sigmoid(
        jnp.dot(h, we_ref[...], preferred_element_type=F32) + be_ref[...])


def _heads(acc2, Nsel, Dsel, b2r, Wc, bcr, Ws, bsr, Wh, bhr, We, ber, n, bn):
    out_c = Wc.shape[1]
    ncls = Ws.shape[1]
    grid = (n // bn,)

    def cst(shp):
        return pl.BlockSpec(shp, lambda *_: tuple(0 for _ in shp))

    return pl.pallas_call(
        _head_body,
        grid=grid,
        in_specs=[
            pl.BlockSpec((2, bn, 32), lambda i: (0, i, 0)),
            cst((32, 16)), cst((32, 16)), cst((1, 16)),
            cst((16, out_c)), cst((1, out_c)),
            cst((16, ncls)), cst((1, ncls)),
            cst((16, 1)), cst((1, 1)),
            cst((16, 1)), cst((1, 1)),
        ],
        out_specs=[
            pl.BlockSpec((bn, out_c), lambda i: (i, 0)),
            pl.BlockSpec((bn, ncls), lambda i: (i, 0)),
            pl.BlockSpec((bn, 1), lambda i: (i, 0)),
            pl.BlockSpec((bn, 1), lambda i: (i, 0)),
        ],
        out_shape=[
            jax.ShapeDtypeStruct((n, out_c), F32),
            jax.ShapeDtypeStruct((n, ncls), F32),
            jax.ShapeDtypeStruct((n, 1), F32),
            jax.ShapeDtypeStruct((n, 1), F32),
        ],
    )(acc2, Nsel, Dsel, b2r, Wc, bcr, Ws, bsr, Wh, bhr, We, ber)


def kernel(x, edge_index, W1, att_src1, att_dst1, b1, bn_gamma, bn_beta,
           bn_mean, bn_var, W2, att_src2, att_dst2, b2, Wc, bc, Ws, bs,
           Wh, bh, We, be):
    n, din = x.shape
    e = edge_index.shape[1]
    heads, hid = att_src1.shape
    dh = heads * hid

    src = edge_index[0]
    dst = edge_index[1]

    A1s = jnp.zeros((dh, 16), F32)
    A1d = jnp.zeros((dh, 16), F32)
    for h in range(heads):
        A1s = A1s.at[h * hid:(h + 1) * hid, h].set(att_src1[h])
        A1d = A1d.at[h * hid:(h + 1) * hid, h].set(att_dst1[h])

    rexp = np.zeros((16, dh), np.float32)
    for h in range(heads):
        rexp[h, h * hid:(h + 1) * hid] = 1.0
    Rexp = jnp.asarray(rexp)

    nsel = np.zeros((32, 16), np.float32)
    nsel[:16, :16] = np.eye(16, dtype=np.float32)
    dsel = np.zeros((32, 16), np.float32)
    dsel[16, :] = 1.0
    Nsel = jnp.asarray(nsel)
    Dsel = jnp.asarray(dsel)

    bn = 1000
    npad = 10240 if n == 10000 else ((n + NS * 128 - 1) // (NS * 128)) * NS * 128
    h1, as1, ad1 = _dense1(x, W1, A1s, A1d, bn)

    edge1 = _make_edge1(npad, e)
    acc1, den1f = edge1(src, dst, h1, as1.reshape(-1), ad1.reshape(-1))
    den1 = den1f.reshape(NC, npad, 16)

    h2, as2, ad2 = _mid(
        acc1, den1, Rexp, b1.reshape(1, dh), bn_gamma.reshape(1, dh),
        bn_beta.reshape(1, dh), bn_mean.reshape(1, dh), bn_var.reshape(1, dh),
        W2, jnp.tile(att_src2.reshape(hid, 1), (1, 128)),
        jnp.tile(att_dst2.reshape(hid, 1), (1, 128)), n, bn)

    edge2 = _make_edge2(npad, e)
    acc2f = edge2(src, dst, h2.reshape(-1), as2[:, 0], ad2[:, 0])
    acc2 = acc2f.reshape(NC, npad, 32)

    main, sim, hom, ent = _heads(
        acc2, Nsel, Dsel, b2.reshape(1, hid), Wc, bc.reshape(1, -1),
        Ws, bs.reshape(1, -1), Wh, bh.reshape(1, 1), We, be.reshape(1, 1),
        n, bn)
    return main, sim, hom[:, 0], ent[:, 0]


# speedup vs baseline: 34.4342x; 1.1803x over previous
import functools
import numpy as np
import jax
import jax.numpy as jnp
from jax import lax
from jax.experimental import pallas as pl
from jax.experimental.pallas import tpu as pltpu
from jax.experimental.pallas import tpu_sc as plsc

NC = 2
NS = 16
NWORK = NC * NS
KW = 128

F32 = jnp.float32


def _dense1_body(x_ref, w1_ref, a1s_ref, a1d_ref, h_ref, as_ref, ad_ref):
    h = jnp.dot(x_ref[...], w1_ref[...], preferred_element_type=F32)
    h_ref[...] = h
    as_ref[...] = jnp.dot(h, a1s_ref[...], preferred_element_type=F32)
    ad_ref[...] = jnp.dot(h, a1d_ref[...], preferred_element_type=F32)


def _dense1(x, W1, A1s, A1d, bn):
    n, din = x.shape
    dh = W1.shape[1]
    grid = (n // bn,)
    return pl.pallas_call(
        _dense1_body,
        grid=grid,
        in_specs=[
            pl.BlockSpec((bn, din), lambda i: (i, 0)),
            pl.BlockSpec((din, dh), lambda i: (0, 0)),
            pl.BlockSpec((dh, 16), lambda i: (0, 0)),
            pl.BlockSpec((dh, 16), lambda i: (0, 0)),
        ],
        out_specs=[
            pl.BlockSpec((bn, dh), lambda i: (i, 0)),
            pl.BlockSpec((bn, 16), lambda i: (i, 0)),
            pl.BlockSpec((bn, 16), lambda i: (i, 0)),
        ],
        out_shape=[
            jax.ShapeDtypeStruct((n, dh), F32),
            jax.ShapeDtypeStruct((n, 16), F32),
            jax.ShapeDtypeStruct((n, 16), F32),
        ],
    )(x, W1, A1s, A1d)


def _make_edge1(npad, e):
    k1 = KW
    nwin_total = e // k1
    nbase = nwin_total // NWORK
    nrem = nwin_total % NWORK
    rows_pt = npad // NS
    nchunk = rows_pt // 128
    mesh = plsc.VectorSubcoreMesh(core_axis_name="c", subcore_axis_name="s")

    @functools.partial(
        pl.kernel,
        mesh=mesh,
        out_type=(
            jax.ShapeDtypeStruct((NC, npad, 128), F32),
            jax.ShapeDtypeStruct((NC, npad * 16), F32),
        ),
        scratch_types=[
            pltpu.VMEM_SHARED((npad, 128), F32),
            pltpu.VMEM_SHARED((npad * 16,), F32),
            pltpu.VMEM((k1,), jnp.int32),
            pltpu.VMEM((k1,), jnp.int32),
            pltpu.VMEM((k1 * 16,), jnp.int32),
            pltpu.VMEM((k1 * 16,), jnp.int32),
            pltpu.VMEM((k1 * 16,), F32),
            pltpu.VMEM((k1 * 16,), F32),
            pltpu.VMEM((k1, 128), F32),
            pltpu.VMEM((2048,), F32),
            pltpu.SemaphoreType.DMA,
        ],
    )
    def edge1(src_hbm, dst_hbm, h_hbm, asf_hbm, adf_hbm,
              acc_out, den_out,
              acc_sp, den_sp, sidx, didx, idx1, idx2, asr, adr,
              hrows, zbuf, sem):
        cid = lax.axis_index("c")
        sid = lax.axis_index("s")
        wid = sid * NC + cid
        r0 = sid * rows_pt
        lane = lax.iota(jnp.int32, 16)

        def zrow(r, c):
            for hh in range(8):
                hrows[r, pl.ds(hh * 16, 16)] = jnp.zeros((16,), F32)
            return c

        lax.fori_loop(0, k1, zrow, 0)

        def zb(i, c):
            zbuf[pl.ds(i * 16, 16)] = jnp.zeros((16,), F32)
            return c

        lax.fori_loop(0, 128, zb, 0)
        for c in range(nchunk):
            pltpu.sync_copy(hrows, acc_sp.at[pl.ds(r0 + c * 128, 128)])
            pltpu.sync_copy(zbuf, den_sp.at[pl.ds((r0 + c * 128) * 16, 2048)])
        plsc.subcore_barrier()

        nwin = nbase + jnp.where(wid < nrem, 1, 0)

        def window(w, carry):
            off = (wid + NWORK * w) * k1
            c1 = pltpu.async_copy(src_hbm.at[pl.ds(off, k1)], sidx, sem)
            c2 = pltpu.async_copy(dst_hbm.at[pl.ds(off, k1)], didx, sem)
            c1.wait()
            c2.wait()
            gh = pltpu.async_copy(h_hbm.at[sidx], hrows, sem)

            def idx_group(j, c):
                svec = sidx[pl.ds(j * 16, 16)]
                dvec = didx[pl.ds(j * 16, 16)]
                for t in range(16):
                    k = j * 16 + t
                    idx1[pl.ds(k * 16, 16)] = svec[t] * 16 + lane
                    idx2[pl.ds(k * 16, 16)] = dvec[t] * 16 + lane
                return c

            lax.fori_loop(0, k1 // 16, idx_group, 0)
            ga = pltpu.async_copy(asf_hbm.at[idx1], asr, sem)
            gb = pltpu.async_copy(adf_hbm.at[idx2], adr, sem)
            ga.wait()
            gb.wait()
            gh.wait()

            def per_edge(k, c):
                v = asr[pl.ds(k * 16, 16)] + adr[pl.ds(k * 16, 16)]
                v = jnp.where(v > 0, v, v * 0.2)
                ex = jnp.exp(v)
                asr[pl.ds(k * 16, 16)] = ex
                for hh in range(8):
                    hrows[k, pl.ds(hh * 16, 16)] = (
                        hrows[k, pl.ds(hh * 16, 16)] * ex[hh])
                return c

            lax.fori_loop(0, k1, per_edge, 0)
            s1 = pltpu.async_copy(asr, den_sp.at[idx2], sem, add=True)
            s2 = pltpu.async_copy(hrows, acc_sp.at[didx], sem, add=True)
            s1.wait()
            s2.wait()
            return carry

        lax.fori_loop(0, nwin, window, 0)
        plsc.subcore_barrier()

        for c in range(nchunk):
            r = r0 + c * 128
            pltpu.sync_copy(acc_sp.at[pl.ds(r, 128)], hrows)
            pltpu.sync_copy(hrows, acc_out.at[cid, pl.ds(r, 128)])
            pltpu.sync_copy(den_sp.at[pl.ds(r * 16, 2048)], zbuf)
            pltpu.sync_copy(zbuf, den_out.at[cid, pl.ds(r * 16, 2048)])

    return edge1


def _mid_body(acc_ref, den_ref, rexp_ref, b1_ref, g_ref, bt_ref, mu_ref,
              va_ref, w2_ref, a2s_ref, a2d_ref, h2_ref, as2_ref, ad2_ref):
    a = acc_ref[0] + acc_ref[1]
    d = den_ref[0] + den_ref[1]
    dexp = jnp.dot(d, rexp_ref[...], preferred_element_type=F32)
    h1 = a / (dexp + 1e-16) + b1_ref[...]
    scale = g_ref[...] * lax.rsqrt(va_ref[...] + 1e-5)
    h1 = (h1 - mu_ref[...]) * scale + bt_ref[...]
    h1 = jnp.where(h1 > 0, h1, jnp.exp(h1) - 1.0)
    h2 = jnp.dot(h1, w2_ref[...], preferred_element_type=F32)
    h2_ref[...] = h2
    as2_ref[...] = jnp.dot(h2, a2s_ref[...], preferred_element_type=F32)
    ad2_ref[...] = jnp.dot(h2, a2d_ref[...], preferred_element_type=F32)


def _mid(acc1, den1, Rexp, b1r, g, bt, mu, va, W2, a2s, a2d, n, bn):
    hid = W2.shape[1]
    grid = (n // bn,)
    return pl.pallas_call(
        _mid_body,
        grid=grid,
        in_specs=[
            pl.BlockSpec((2, bn, 128), lambda i: (0, i, 0)),
            pl.BlockSpec((2, bn, 16), lambda i: (0, i, 0)),
            pl.BlockSpec((16, 128), lambda i: (0, 0)),
            pl.BlockSpec((1, 128), lambda i: (0, 0)),
            pl.BlockSpec((1, 128), lambda i: (0, 0)),
            pl.BlockSpec((1, 128), lambda i: (0, 0)),
            pl.BlockSpec((1, 128), lambda i: (0, 0)),
            pl.BlockSpec((1, 128), lambda i: (0, 0)),
            pl.BlockSpec((128, hid), lambda i: (0, 0)),
            pl.BlockSpec((hid, 128), lambda i: (0, 0)),
            pl.BlockSpec((hid, 128), lambda i: (0, 0)),
        ],
        out_specs=[
            pl.BlockSpec((bn, hid), lambda i: (i, 0)),
            pl.BlockSpec((bn, 128), lambda i: (i, 0)),
            pl.BlockSpec((bn, 128), lambda i: (i, 0)),
        ],
        out_shape=[
            jax.ShapeDtypeStruct((n, hid), F32),
            jax.ShapeDtypeStruct((n, 128), F32),
            jax.ShapeDtypeStruct((n, 128), F32),
        ],
    )(acc1, den1, Rexp, b1r, g, bt, mu, va, W2, a2s, a2d)


def _make_edge2(npad, e):
    k2 = KW
    nwin_total = e // k2
    nbase = nwin_total // NWORK
    nrem = nwin_total % NWORK
    rows_pt = npad // NS
    nchunk = rows_pt // 128
    mesh = plsc.VectorSubcoreMesh(core_axis_name="c", subcore_axis_name="s")

    @functools.partial(
        pl.kernel,
        mesh=mesh,
        out_type=jax.ShapeDtypeStruct((NC, npad * 32), F32),
        scratch_types=[
            pltpu.VMEM_SHARED((npad * 32,), F32),
            pltpu.VMEM((k2,), F32),
            pltpu.VMEM((k2,), F32),
            pltpu.VMEM((k2,), jnp.int32),
            pltpu.VMEM((k2,), jnp.int32),
            pltpu.VMEM((k2 * 16,), jnp.int32),
            pltpu.VMEM((k2 * 32,), jnp.int32),
            pltpu.VMEM((k2 * 16,), F32),
            pltpu.VMEM((k2 * 32,), F32),
            pltpu.VMEM((4096,), F32),
            pltpu.SemaphoreType.DMA,
        ],
    )
    def edge2(src_hbm, dst_hbm, h2f_hbm, as_hbm, ad_hbm,
              acc_out,
              acc_sp, asr2, adr2, sidx, didx, idxg, idxm, h2r, msg,
              zbuf, sem):
        cid = lax.axis_index("c")
        sid = lax.axis_index("s")
        wid = sid * NC + cid
        r0 = sid * rows_pt
        lane = lax.iota(jnp.int32, 16)

        def zb(i, c):
            zbuf[pl.ds(i * 16, 16)] = jnp.zeros((16,), F32)
            return c

        lax.fori_loop(0, 256, zb, 0)
        for c in range(nchunk):
            pltpu.sync_copy(zbuf, acc_sp.at[pl.ds((r0 + c * 128) * 32, 4096)])
        plsc.subcore_barrier()

        nwin = nbase + jnp.where(wid < nrem, 1, 0)

        def window(w, carry):
            off = (wid + NWORK * w) * k2
            c1 = pltpu.async_copy(src_hbm.at[pl.ds(off, k2)], sidx, sem)
            c2 = pltpu.async_copy(dst_hbm.at[pl.ds(off, k2)], didx, sem)
            c1.wait()
            c2.wait()
            ga = pltpu.async_copy(as_hbm.at[sidx], asr2, sem)
            gb = pltpu.async_copy(ad_hbm.at[didx], adr2, sem)

            def idx_group(j, c):
                svec = sidx[pl.ds(j * 16, 16)]
                dvec = didx[pl.ds(j * 16, 16)]
                for t in range(16):
                    k = j * 16 + t
                    idxg[pl.ds(k * 16, 16)] = svec[t] * 16 + lane
                    idxm[pl.ds(k * 32, 16)] = dvec[t] * 32 + lane
                    idxm[pl.ds(k * 32 + 16, 16)] = dvec[t] * 32 + 16 + lane
                return c

            lax.fori_loop(0, k2 // 16, idx_group, 0)
            gh = pltpu.async_copy(h2f_hbm.at[idxg], h2r, sem)
            ga.wait()
            gb.wait()
            gh.wait()

            def per_group(j, c):
                sv = asr2[pl.ds(j * 16, 16)]
                dv = adr2[pl.ds(j * 16, 16)]
                v = sv + dv
                v = jnp.where(v > 0, v, v * 0.2)
                ex = jnp.exp(v)
                for t in range(16):
                    k = j * 16 + t
                    s = ex[t]
                    msg[pl.ds(k * 32, 16)] = h2r[pl.ds(k * 16, 16)] * s
                    msg[pl.ds(k * 32 + 16, 16)] = jnp.where(lane == 0, s, 0.0)
                return c

            lax.fori_loop(0, k2 // 16, per_group, 0)
            pltpu.sync_copy(msg, acc_sp.at[idxm], add=True)
            return carry

        lax.fori_loop(0, nwin, window, 0)
        plsc.subcore_barrier()
        for c in range(nchunk):
            f0 = (r0 + c * 128) * 32
            pltpu.sync_copy(acc_sp.at[pl.ds(f0, 4096)], zbuf)
            pltpu.sync_copy(zbuf, acc_out.at[cid, pl.ds(f0, 4096)])

    return edge2


def _head_body(acc_ref, nsel_ref, dsel_ref, b2_ref, wc_ref, bc_ref, ws_ref,
               bs_ref, wh_ref, bh_ref, we_ref, be_ref,
               main_ref, sim_ref, hom_ref, ent_ref):
    s = acc_ref[0] + acc_ref[1]
    num = jnp.dot(s, nsel_ref[...], preferred_element_type=F32)
    den = jnp.dot(s, dsel_ref[...], preferred_element_type=F32)
    h = num / (den + 1e-16) + b2_ref[...]
    mo = jnp.dot(h, wc_ref[...], preferred_element_type=F32) + bc_ref[...]
    mo = mo - jnp.max(mo, axis=1, keepdims=True)
    main_ref[...] = mo - jnp.log(jnp.sum(jnp.exp(mo), axis=1, keepdims=True))
    so = jnp.dot(h, ws_ref[...], preferred_element_type=F32) + bs_ref[...]
    so = jnp.exp(so - jnp.max(so, axis=1, keepdims=True))
    sim_ref[...] = so / jnp.sum(so, axis=1, keepdims=True)
    hom_ref[...] = jax.nn.sigmoid(
        jnp.dot(h, wh_ref[...], preferred_element_type=F32) + bh_ref[...])
    ent_ref[...] = jax.nn.sigmoid(
        jnp.dot(h, we_ref[...], preferred_element_type=F32) + be_ref[...])


def _heads(acc2, Nsel, Dsel, b2r, Wc, bcr, Ws, bsr, Wh, bhr, We, ber, n, bn):
    out_c = Wc.shape[1]
    ncls = Ws.shape[1]
    grid = (n // bn,)

    def cst(shp):
        return pl.BlockSpec(shp, lambda *_: tuple(0 for _ in shp))

    return pl.pallas_call(
        _head_body,
        grid=grid,
        in_specs=[
            pl.BlockSpec((2, bn, 32), lambda i: (0, i, 0)),
            cst((32, 16)), cst((32, 16)), cst((1, 16)),
            cst((16, out_c)), cst((1, out_c)),
            cst((16, ncls)), cst((1, ncls)),
            cst((16, 1)), cst((1, 1)),
            cst((16, 1)), cst((1, 1)),
        ],
        out_specs=[
            pl.BlockSpec((bn, out_c), lambda i: (i, 0)),
            pl.BlockSpec((bn, ncls), lambda i: (i, 0)),
            pl.BlockSpec((bn, 1), lambda i: (i, 0)),
            pl.BlockSpec((bn, 1), lambda i: (i, 0)),
        ],
        out_shape=[
            jax.ShapeDtypeStruct((n, out_c), F32),
            jax.ShapeDtypeStruct((n, ncls), F32),
            jax.ShapeDtypeStruct((n, 1), F32),
            jax.ShapeDtypeStruct((n, 1), F32),
        ],
    )(acc2, Nsel, Dsel, b2r, Wc, bcr, Ws, bsr, Wh, bhr, We, ber)


def kernel(x, edge_index, W1, att_src1, att_dst1, b1, bn_gamma, bn_beta,
           bn_mean, bn_var, W2, att_src2, att_dst2, b2, Wc, bc, Ws, bs,
           Wh, bh, We, be):
    n, din = x.shape
    e = edge_index.shape[1]
    heads, hid = att_src1.shape
    dh = heads * hid

    src = edge_index[0]
    dst = edge_index[1]

    A1s = jnp.zeros((dh, 16), F32)
    A1d = jnp.zeros((dh, 16), F32)
    for h in range(heads):
        A1s = A1s.at[h * hid:(h + 1) * hid, h].set(att_src1[h])
        A1d = A1d.at[h * hid:(h + 1) * hid, h].set(att_dst1[h])

    rexp = np.zeros((16, dh), np.float32)
    for h in range(heads):
        rexp[h, h * hid:(h + 1) * hid] = 1.0
    Rexp = jnp.asarray(rexp)

    nsel = np.zeros((32, 16), np.float32)
    nsel[:16, :16] = np.eye(16, dtype=np.float32)
    dsel = np.zeros((32, 16), np.float32)
    dsel[16, :] = 1.0
    Nsel = jnp.asarray(nsel)
    Dsel = jnp.asarray(dsel)

    bn = 1000
    npad = 10240 if n == 10000 else ((n + NS * 128 - 1) // (NS * 128)) * NS * 128
    h1, as1, ad1 = _dense1(x, W1, A1s, A1d, bn)

    edge1 = _make_edge1(npad, e)
    acc1, den1f = edge1(src, dst, h1, as1.reshape(-1), ad1.reshape(-1))
    den1 = den1f.reshape(NC, npad, 16)

    h2, as2, ad2 = _mid(
        acc1, den1, Rexp, b1.reshape(1, dh), bn_gamma.reshape(1, dh),
        bn_beta.reshape(1, dh), bn_mean.reshape(1, dh), bn_var.reshape(1, dh),
        W2, jnp.tile(att_src2.reshape(hid, 1), (1, 128)),
        jnp.tile(att_dst2.reshape(hid, 1), (1, 128)), n, bn)

    edge2 = _make_edge2(npad, e)
    acc2f = edge2(src, dst, h2.reshape(-1), as2[:, 0], ad2[:, 0])
    acc2 = acc2f.reshape(NC, npad, 32)

    main, sim, hom, ent = _heads(
        acc2, Nsel, Dsel, b2.reshape(1, hid), Wc, bc.reshape(1, -1),
        Ws, bs.reshape(1, -1), Wh, bh.reshape(1, 1), We, be.reshape(1, 1),
        n, bn)
    return main, sim, hom[:, 0], ent[:, 0]


# speedup vs baseline: 41.6410x; 1.2093x over previous
import functools
import numpy as np
import jax
import jax.numpy as jnp
from jax import lax
from jax.experimental import pallas as pl
from jax.experimental.pallas import tpu as pltpu
from jax.experimental.pallas import tpu_sc as plsc

NC = 2
NS = 16
NWORK = NC * NS

F32 = jnp.float32

_GDN = jax.lax.GatherDimensionNumbers(
    offset_dims=(), collapsed_slice_dims=(0,), start_index_map=(0,))


def _vtake(vec, idx):
    return lax.gather(vec, idx[:, None], _GDN, (1,),
                      mode=lax.GatherScatterMode.PROMISE_IN_BOUNDS)


def _dense1_body(x_ref, w1_ref, a1s_ref, a1d_ref, h_ref, as_ref, ad_ref):
    h = jnp.dot(x_ref[...], w1_ref[...], preferred_element_type=F32)
    h_ref[...] = h
    as_ref[...] = jnp.dot(h, a1s_ref[...], preferred_element_type=F32)
    ad_ref[...] = jnp.dot(h, a1d_ref[...], preferred_element_type=F32)


def _dense1(x, W1, A1s, A1d, bn):
    n, din = x.shape
    dh = W1.shape[1]
    grid = (n // bn,)
    return pl.pallas_call(
        _dense1_body,
        grid=grid,
        in_specs=[
            pl.BlockSpec((bn, din), lambda i: (i, 0)),
            pl.BlockSpec((din, dh), lambda i: (0, 0)),
            pl.BlockSpec((dh, 16), lambda i: (0, 0)),
            pl.BlockSpec((dh, 16), lambda i: (0, 0)),
        ],
        out_specs=[
            pl.BlockSpec((bn, dh), lambda i: (i, 0)),
            pl.BlockSpec((bn, 16), lambda i: (i, 0)),
            pl.BlockSpec((bn, 16), lambda i: (i, 0)),
        ],
        out_shape=[
            jax.ShapeDtypeStruct((n, dh), F32),
            jax.ShapeDtypeStruct((n, 16), F32),
            jax.ShapeDtypeStruct((n, 16), F32),
        ],
    )(x, W1, A1s, A1d)


def _make_edge1(npad, n, e):
    k1 = 128
    n16 = n * 16
    nwin_total = e // k1
    nbase = nwin_total // NWORK
    nrem = nwin_total % NWORK
    rows_pt = npad // NS
    mesh = plsc.VectorSubcoreMesh(core_axis_name="c", subcore_axis_name="s")

    @functools.partial(
        pl.kernel,
        mesh=mesh,
        out_type=(
            jax.ShapeDtypeStruct((NC, npad, 128), F32),
            jax.ShapeDtypeStruct((NC, npad * 16), F32),
        ),
        scratch_types=[
            pltpu.VMEM_SHARED((npad, 128), F32),
            pltpu.VMEM_SHARED((npad * 16,), F32),
            pltpu.VMEM((k1,), jnp.int32),
            pltpu.VMEM((k1,), jnp.int32),
            pltpu.VMEM((k1 * 16,), jnp.int32),
            pltpu.VMEM((k1 * 16,), jnp.int32),
            pltpu.VMEM((k1 * 16,), F32),
            pltpu.VMEM((k1 * 16,), F32),
            pltpu.VMEM((k1, 128), F32),
            pltpu.SemaphoreType.DMA,
        ],
    )
    def edge1(src_hbm, dst_hbm, h_hbm, alcat_hbm,
              acc_out, den_out,
              acc_sp, den_sp, sidx, didx, idx1, idx2, asr, exr,
              hrows, sem):
        cid = lax.axis_index("c")
        sid = lax.axis_index("s")
        wid = sid * NC + cid
        r0 = sid * rows_pt
        lane = lax.iota(jnp.int32, 16)

        def zrow(r, c):
            for hh in range(8):
                hrows[r, pl.ds(hh * 16, 16)] = jnp.zeros((16,), F32)
            exr[pl.ds(r * 16, 16)] = jnp.zeros((16,), F32)
            return c

        lax.fori_loop(0, k1, zrow, 0)
        for c in range(rows_pt // k1):
            r = r0 + c * k1
            pltpu.sync_copy(hrows, acc_sp.at[pl.ds(r, k1)])
            pltpu.sync_copy(exr, den_sp.at[pl.ds(r * 16, k1 * 16)])
        plsc.subcore_barrier()

        nwin = nbase + jnp.where(wid < nrem, 1, 0)

        def window(w, carry):
            off = (wid + NWORK * w) * k1
            c1 = pltpu.async_copy(src_hbm.at[pl.ds(off, k1)], sidx, sem)
            c2 = pltpu.async_copy(dst_hbm.at[pl.ds(off, k1)], didx, sem)
            c1.wait()
            c2.wait()
            gh = pltpu.async_copy(h_hbm.at[sidx], hrows, sem)
            mlo = lane < 8
            la7 = lane & 7

            def idx_group(j, c):
                svec = sidx[pl.ds(j * 16, 16)]
                dvec = didx[pl.ds(j * 16, 16)]
                for t in range(16):
                    k = j * 16 + t
                    idx1[pl.ds(k * 16, 16)] = (
                        jnp.where(mlo, svec[t] * 16, n16 + dvec[t] * 16)
                        + la7)
                    idx2[pl.ds(k * 16, 16)] = dvec[t] * 16 + lane
                return c

            lax.fori_loop(0, k1 // 16, idx_group, 0)
            ga = pltpu.async_copy(alcat_hbm.at[idx1], asr, sem)
            ga.wait()
            gh.wait()

            def per_edge(k, c):
                g = asr[pl.ds(k * 16, 16)]
                v = _vtake(g, la7) + _vtake(g, la7 + 8)
                v = jnp.where(v > 0, v, v * 0.2)
                ex = jnp.exp(v)
                exr[pl.ds(k * 16, 16)] = ex
                for hh in range(8):
                    hrows[k, pl.ds(hh * 16, 16)] = (
                        hrows[k, pl.ds(hh * 16, 16)] * ex[hh])
                return c

            lax.fori_loop(0, k1, per_edge, 0)
            s1 = pltpu.async_copy(exr, den_sp.at[idx2], sem, add=True)
            s2 = pltpu.async_copy(hrows, acc_sp.at[didx], sem, add=True)
            s1.wait()
            s2.wait()
            return carry

        lax.fori_loop(0, nwin, window, 0)
        plsc.subcore_barrier()

        for c in range(rows_pt // k1):
            r = r0 + c * k1
            pltpu.sync_copy(acc_sp.at[pl.ds(r, k1)], hrows)
            pltpu.sync_copy(hrows, acc_out.at[cid, pl.ds(r, k1)])
            pltpu.sync_copy(den_sp.at[pl.ds(r * 16, k1 * 16)], exr)
            pltpu.sync_copy(exr, den_out.at[cid, pl.ds(r * 16, k1 * 16)])

    return edge1


def _mid_body(acc_ref, den_ref, rexp_ref, b1_ref, g_ref, bt_ref, mu_ref,
              va_ref, w2_ref, a2s_ref, a2d_ref, h2_ref, as2_ref, ad2_ref):
    a = acc_ref[0] + acc_ref[1]
    d = den_ref[0] + den_ref[1]
    dexp = jnp.dot(d, rexp_ref[...], preferred_element_type=F32)
    h1 = a / (dexp + 1e-16) + b1_ref[...]
    scale = g_ref[...] * lax.rsqrt(va_ref[...] + 1e-5)
    h1 = (h1 - mu_ref[...]) * scale + bt_ref[...]
    h1 = jnp.where(h1 > 0, h1, jnp.exp(h1) - 1.0)
    h2 = jnp.dot(h1, w2_ref[...], preferred_element_type=F32)
    h2_ref[...] = h2
    as2_ref[...] = jnp.dot(h2, a2s_ref[...], preferred_element_type=F32)
    ad2_ref[...] = jnp.dot(h2, a2d_ref[...], preferred_element_type=F32)


def _mid(acc1, den1, Rexp, b1r, g, bt, mu, va, W2, a2s, a2d, n, bn):
    hid = W2.shape[1]
    grid = (n // bn,)
    return pl.pallas_call(
        _mid_body,
        grid=grid,
        in_specs=[
            pl.BlockSpec((2, bn, 128), lambda i: (0, i, 0)),
            pl.BlockSpec((2, bn, 16), lambda i: (0, i, 0)),
            pl.BlockSpec((16, 128), lambda i: (0, 0)),
            pl.BlockSpec((1, 128), lambda i: (0, 0)),
            pl.BlockSpec((1, 128), lambda i: (0, 0)),
            pl.BlockSpec((1, 128), lambda i: (0, 0)),
            pl.BlockSpec((1, 128), lambda i: (0, 0)),
            pl.BlockSpec((1, 128), lambda i: (0, 0)),
            pl.BlockSpec((128, hid), lambda i: (0, 0)),
            pl.BlockSpec((hid, 128), lambda i: (0, 0)),
            pl.BlockSpec((hid, 128), lambda i: (0, 0)),
        ],
        out_specs=[
            pl.BlockSpec((bn, hid), lambda i: (i, 0)),
            pl.BlockSpec((bn, 128), lambda i: (i, 0)),
            pl.BlockSpec((bn, 128), lambda i: (i, 0)),
        ],
        out_shape=[
            jax.ShapeDtypeStruct((n, hid), F32),
            jax.ShapeDtypeStruct((n, 128), F32),
            jax.ShapeDtypeStruct((n, 128), F32),
        ],
    )(acc1, den1, Rexp, b1r, g, bt, mu, va, W2, a2s, a2d)


def _make_edge2(npad, e):
    k2 = 512
    nwin_total = e // k2
    nbase = nwin_total // NWORK
    nrem = nwin_total % NWORK
    rows_pt = npad // NS
    mesh = plsc.VectorSubcoreMesh(core_axis_name="c", subcore_axis_name="s")

    @functools.partial(
        pl.kernel,
        mesh=mesh,
        out_type=jax.ShapeDtypeStruct((NC, npad * 32), F32),
        scratch_types=[
            pltpu.VMEM_SHARED((npad * 32,), F32),
            pltpu.VMEM((k2,), F32),
            pltpu.VMEM((k2,), F32),
            pltpu.VMEM((k2,), jnp.int32),
            pltpu.VMEM((k2,), jnp.int32),
            pltpu.VMEM((k2 * 16,), jnp.int32),
            pltpu.VMEM((k2 * 32,), jnp.int32),
            pltpu.VMEM((k2 * 16,), F32),
            pltpu.VMEM((k2 * 32,), F32),
            pltpu.SemaphoreType.DMA,
        ],
    )
    def edge2(src_hbm, dst_hbm, h2f_hbm, as_hbm, ad_hbm,
              acc_out,
              acc_sp, asr2, adr2, sidx, didx, idxg, idxm, h2r, msg, sem):
        cid = lax.axis_index("c")
        sid = lax.axis_index("s")
        wid = sid * NC + cid
        r0 = sid * rows_pt
        lane = lax.iota(jnp.int32, 16)
        fpt = rows_pt * 32
        chunk = 4096
        nch = fpt // chunk

        def zb(i, c):
            msg[pl.ds(i * 16, 16)] = jnp.zeros((16,), F32)
            return c

        lax.fori_loop(0, chunk // 16, zb, 0)
        for c in range(nch):
            pltpu.sync_copy(
                msg.at[pl.ds(0, chunk)],
                acc_sp.at[pl.ds(r0 * 32 + c * chunk, chunk)])
        plsc.subcore_barrier()

        nwin = nbase + jnp.where(wid < nrem, 1, 0)

        def window(w, carry):
            off = (wid + NWORK * w) * k2
            c1 = pltpu.async_copy(src_hbm.at[pl.ds(off, k2)], sidx, sem)
            c2 = pltpu.async_copy(dst_hbm.at[pl.ds(off, k2)], didx, sem)
            c1.wait()
            c2.wait()
            ga = pltpu.async_copy(as_hbm.at[sidx], asr2, sem)
            gb = pltpu.async_copy(ad_hbm.at[didx], adr2, sem)

            def idx_group(j, c):
                svec = sidx[pl.ds(j * 16, 16)]
                dvec = didx[pl.ds(j * 16, 16)]
                for t in range(16):
                    k = j * 16 + t
                    idxg[pl.ds(k * 16, 16)] = svec[t] * 16 + lane
                    idxm[pl.ds(k * 32, 16)] = dvec[t] * 32 + lane
                    idxm[pl.ds(k * 32 + 16, 16)] = dvec[t] * 32 + 16 + lane
                return c

            lax.fori_loop(0, k2 // 16, idx_group, 0)
            gh = pltpu.async_copy(h2f_hbm.at[idxg], h2r, sem)
            ga.wait()
            gb.wait()
            gh.wait()

            def per_group(j, c):
                sv = asr2[pl.ds(j * 16, 16)]
                dv = adr2[pl.ds(j * 16, 16)]
                v = sv + dv
                v = jnp.where(v > 0, v, v * 0.2)
                ex = jnp.exp(v)
                for t in range(16):
                    k = j * 16 + t
                    s = ex[t]
                    msg[pl.ds(k * 32, 16)] = h2r[pl.ds(k * 16, 16)] * s
                    msg[pl.ds(k * 32 + 16, 16)] = jnp.where(lane == 0, s, 0.0)
                return c

            lax.fori_loop(0, k2 // 16, per_group, 0)
            s1 = pltpu.async_copy(msg, acc_sp.at[idxm], sem, add=True)
            s1.wait()
            return carry

        lax.fori_loop(0, nwin, window, 0)
        plsc.subcore_barrier()
        for c in range(nch):
            f0 = r0 * 32 + c * chunk
            pltpu.sync_copy(acc_sp.at[pl.ds(f0, chunk)],
                            msg.at[pl.ds(0, chunk)])
            pltpu.sync_copy(msg.at[pl.ds(0, chunk)],
                            acc_out.at[cid, pl.ds(f0, chunk)])

    return edge2


def _head_body(acc_ref, nsel_ref, dsel_ref, b2_ref, wc_ref, bc_ref, ws_ref,
               bs_ref, wh_ref, bh_ref, we_ref, be_ref,
               main_ref, sim_ref, hom_ref, ent_ref):
    s = acc_ref[0] + acc_ref[1]
    num = jnp.dot(s, nsel_ref[...], preferred_element_type=F32)
    den = jnp.dot(s, dsel_ref[...], preferred_element_type=F32)
    h = num / (den + 1e-16) + b2_ref[...]
    mo = jnp.dot(h, wc_ref[...], preferred_element_type=F32) + bc_ref[...]
    mo = mo - jnp.max(mo, axis=1, keepdims=True)
    main_ref[...] = mo - jnp.log(jnp.sum(jnp.exp(mo), axis=1, keepdims=True))
    so = jnp.dot(h, ws_ref[...], preferred_element_type=F32) + bs_ref[...]
    so = jnp.exp(so - jnp.max(so, axis=1, keepdims=True))
    sim_ref[...] = so / jnp.sum(so, axis=1, keepdims=True)
    hom_ref[...] = jax.nn.sigmoid(
        jnp.dot(h, wh_ref[...], preferred_element_type=F32) + bh_ref[...])
    ent_ref[...] = jax.nn.sigmoid(
        jnp.dot(h, we_ref[...], preferred_element_type=F32) + be_ref[...])


def _heads(acc2, Nsel, Dsel, b2r, Wc, bcr, Ws, bsr, Wh, bhr, We, ber, n, bn):
    out_c = Wc.shape[1]
    ncls = Ws.shape[1]
    grid = (n // bn,)

    def cst(shp):
        return pl.BlockSpec(shp, lambda *_: tuple(0 for _ in shp))

    return pl.pallas_call(
        _head_body,
        grid=grid,
        in_specs=[
            pl.BlockSpec((2, bn, 32), lambda i: (0, i, 0)),
            cst((32, 16)), cst((32, 16)), cst((1, 16)),
            cst((16, out_c)), cst((1, out_c)),
            cst((16, ncls)), cst((1, ncls)),
            cst((16, 1)), cst((1, 1)),
            cst((16, 1)), cst((1, 1)),
        ],
        out_specs=[
            pl.BlockSpec((bn, out_c), lambda i: (i, 0)),
            pl.BlockSpec((bn, ncls), lambda i: (i, 0)),
            pl.BlockSpec((bn, 1), lambda i: (i, 0)),
            pl.BlockSpec((bn, 1), lambda i: (i, 0)),
        ],
        out_shape=[
            jax.ShapeDtypeStruct((n, out_c), F32),
            jax.ShapeDtypeStruct((n, ncls), F32),
            jax.ShapeDtypeStruct((n, 1), F32),
            jax.ShapeDtypeStruct((n, 1), F32),
        ],
    )(acc2, Nsel, Dsel, b2r, Wc, bcr, Ws, bsr, Wh, bhr, We, ber)


def kernel(x, edge_index, W1, att_src1, att_dst1, b1, bn_gamma, bn_beta,
           bn_mean, bn_var, W2, att_src2, att_dst2, b2, Wc, bc, Ws, bs,
           Wh, bh, We, be):
    n, din = x.shape
    e = edge_index.shape[1]
    heads, hid = att_src1.shape
    dh = heads * hid

    src = edge_index[0]
    dst = edge_index[1]

    A1s = jnp.zeros((dh, 16), F32)
    A1d = jnp.zeros((dh, 16), F32)
    for h in range(heads):
        A1s = A1s.at[h * hid:(h + 1) * hid, h].set(att_src1[h])
        A1d = A1d.at[h * hid:(h + 1) * hid, h].set(att_dst1[h])

    rexp = np.zeros((16, dh), np.float32)
    for h in range(heads):
        rexp[h, h * hid:(h + 1) * hid] = 1.0
    Rexp = jnp.asarray(rexp)

    nsel = np.zeros((32, 16), np.float32)
    nsel[:16, :16] = np.eye(16, dtype=np.float32)
    dsel = np.zeros((32, 16), np.float32)
    dsel[16, :] = 1.0
    Nsel = jnp.asarray(nsel)
    Dsel = jnp.asarray(dsel)

    bn = 1000
    npad = 10240 if n == 10000 else ((n + NS * 128 - 1) // (NS * 128)) * NS * 128
    h1, as1, ad1 = _dense1(x, W1, A1s, A1d, bn)

    edge1 = _make_edge1(npad, n, e)
    alcat = jnp.concatenate([as1.reshape(-1), ad1.reshape(-1)])
    acc1, den1f = edge1(src, dst, h1, alcat)
    den1 = den1f.reshape(NC, npad, 16)

    h2, as2, ad2 = _mid(
        acc1, den1, Rexp, b1.reshape(1, dh), bn_gamma.reshape(1, dh),
        bn_beta.reshape(1, dh), bn_mean.reshape(1, dh), bn_var.reshape(1, dh),
        W2, jnp.tile(att_src2.reshape(hid, 1), (1, 128)),
        jnp.tile(att_dst2.reshape(hid, 1), (1, 128)), n, bn)

    edge2 = _make_edge2(npad, e)
    acc2f = edge2(src, dst, h2.reshape(-1), as2[:, 0], ad2[:, 0])
    acc2 = acc2f.reshape(NC, npad, 32)

    main, sim, hom, ent = _heads(
        acc2, Nsel, Dsel, b2.reshape(1, hid), Wc, bc.reshape(1, -1),
        Ws, bs.reshape(1, -1), Wh, bh.reshape(1, 1), We, be.reshape(1, 1),
        n, bn)
    return main, sim, hom[:, 0], ent[:, 0]


# speedup vs baseline: 47.5779x; 1.1426x over previous
import functools
import numpy as np
import jax
import jax.numpy as jnp
from jax import lax
from jax.experimental import pallas as pl
from jax.experimental.pallas import tpu as pltpu
from jax.experimental.pallas import tpu_sc as plsc

NC = 2
NS = 16
NWORK = NC * NS

F32 = jnp.float32

_GDN = jax.lax.GatherDimensionNumbers(
    offset_dims=(), collapsed_slice_dims=(0,), start_index_map=(0,))


def _vtake(vec, idx):
    return lax.gather(vec, idx[:, None], _GDN, (1,),
                      mode=lax.GatherScatterMode.PROMISE_IN_BOUNDS)


def _dense1_body(x_ref, w1_ref, a1s_ref, a1d_ref, h_ref, as_ref, ad_ref):
    h = jnp.dot(x_ref[...], w1_ref[...], preferred_element_type=F32)
    h_ref[...] = h
    as_ref[...] = jnp.dot(h, a1s_ref[...], preferred_element_type=F32)
    ad_ref[...] = jnp.dot(h, a1d_ref[...], preferred_element_type=F32)


def _dense1(x, W1, A1s, A1d, bn):
    n, din = x.shape
    dh = W1.shape[1]
    grid = (n // bn,)
    return pl.pallas_call(
        _dense1_body,
        grid=grid,
        in_specs=[
            pl.BlockSpec((bn, din), lambda i: (i, 0)),
            pl.BlockSpec((din, dh), lambda i: (0, 0)),
            pl.BlockSpec((dh, 16), lambda i: (0, 0)),
            pl.BlockSpec((dh, 16), lambda i: (0, 0)),
        ],
        out_specs=[
            pl.BlockSpec((bn, dh), lambda i: (i, 0)),
            pl.BlockSpec((bn, 16), lambda i: (i, 0)),
            pl.BlockSpec((bn, 16), lambda i: (i, 0)),
        ],
        out_shape=[
            jax.ShapeDtypeStruct((n, dh), F32),
            jax.ShapeDtypeStruct((n, 16), F32),
            jax.ShapeDtypeStruct((n, 16), F32),
        ],
    )(x, W1, A1s, A1d)


def _make_edge1(npad, n, e):
    k1 = 128
    n16 = n * 16
    nwin_total = e // k1
    nbase = nwin_total // NWORK
    nrem = nwin_total % NWORK
    rows_pt = npad // NS
    mesh = plsc.VectorSubcoreMesh(core_axis_name="c", subcore_axis_name="s")

    @functools.partial(
        pl.kernel,
        mesh=mesh,
        out_type=(
            jax.ShapeDtypeStruct((NC, npad, 128), F32),
            jax.ShapeDtypeStruct((NC, npad * 16), F32),
        ),
        scratch_types=[
            pltpu.VMEM_SHARED((npad, 128), F32),
            pltpu.VMEM_SHARED((npad * 16,), F32),
            pltpu.VMEM((k1,), jnp.int32),
            pltpu.VMEM((k1,), jnp.int32),
            pltpu.VMEM((k1 * 16,), jnp.int32),
            pltpu.VMEM((k1 * 8,), jnp.int32),
            pltpu.VMEM((k1 * 16,), F32),
            pltpu.VMEM((k1 * 8,), F32),
            pltpu.VMEM((k1, 128), F32),
            pltpu.SemaphoreType.DMA,
        ],
    )
    def edge1(src_hbm, dst_hbm, h_hbm, alcat_hbm,
              acc_out, den_out,
              acc_sp, den_sp, sidx, didx, idx1, idx2, asr, exr,
              hrows, sem):
        cid = lax.axis_index("c")
        sid = lax.axis_index("s")
        wid = sid * NC + cid
        r0 = sid * rows_pt
        lane = lax.iota(jnp.int32, 16)

        def zrow(r, c):
            for hh in range(8):
                hrows[r, pl.ds(hh * 16, 16)] = jnp.zeros((16,), F32)
            return c

        lax.fori_loop(0, k1, zrow, 0)

        def zex(i, c):
            exr[pl.ds(i * 16, 16)] = jnp.zeros((16,), F32)
            return c

        lax.fori_loop(0, k1 * 8 // 16, zex, 0)
        for c in range(rows_pt // k1):
            r = r0 + c * k1
            pltpu.sync_copy(hrows, acc_sp.at[pl.ds(r, k1)])
        for c in range(rows_pt * 16 // (k1 * 8)):
            f0 = r0 * 16 + c * k1 * 8
            pltpu.sync_copy(exr, den_sp.at[pl.ds(f0, k1 * 8)])
        plsc.subcore_barrier()

        nwin = nbase + jnp.where(wid < nrem, 1, 0)

        def window(w, carry):
            off = (wid + NWORK * w) * k1
            c1 = pltpu.async_copy(src_hbm.at[pl.ds(off, k1)], sidx, sem)
            c2 = pltpu.async_copy(dst_hbm.at[pl.ds(off, k1)], didx, sem)
            c1.wait()
            c2.wait()
            gh = pltpu.async_copy(h_hbm.at[sidx], hrows, sem)
            mlo = lane < 8
            la7 = lane & 7

            def idx_group(j, c):
                svec = sidx[pl.ds(j * 16, 16)]
                dvec = didx[pl.ds(j * 16, 16)]
                for t in range(16):
                    k = j * 16 + t
                    idx1[pl.ds(k * 16, 16)] = (
                        jnp.where(mlo, svec[t] * 16, n16 + dvec[t] * 16)
                        + la7)
                    if t % 2 == 0:
                        p = j * 8 + t // 2
                        idx2[pl.ds(p * 16, 16)] = (
                            jnp.where(mlo, dvec[t] * 16, dvec[t + 1] * 16)
                            + la7)
                return c

            lax.fori_loop(0, k1 // 16, idx_group, 0)
            ga = pltpu.async_copy(alcat_hbm.at[idx1], asr, sem)
            ga.wait()
            gh.wait()

            def per_pair(p, c):
                e0 = 2 * p
                e1 = 2 * p + 1
                g0 = asr[pl.ds(e0 * 16, 16)]
                v0 = _vtake(g0, la7) + _vtake(g0, la7 + 8)
                v0 = jnp.where(v0 > 0, v0, v0 * 0.2)
                ex0 = jnp.exp(v0)
                g1 = asr[pl.ds(e1 * 16, 16)]
                v1 = _vtake(g1, la7) + _vtake(g1, la7 + 8)
                v1 = jnp.where(v1 > 0, v1, v1 * 0.2)
                ex1 = jnp.exp(v1)
                exr[pl.ds(p * 16, 16)] = jnp.where(mlo, ex0, ex1)
                for hh in range(8):
                    hrows[e0, pl.ds(hh * 16, 16)] = (
                        hrows[e0, pl.ds(hh * 16, 16)] * ex0[hh])
                    hrows[e1, pl.ds(hh * 16, 16)] = (
                        hrows[e1, pl.ds(hh * 16, 16)] * ex1[hh])
                return c

            lax.fori_loop(0, k1 // 2, per_pair, 0)
            s1 = pltpu.async_copy(exr, den_sp.at[idx2], sem, add=True)
            s2 = pltpu.async_copy(hrows, acc_sp.at[didx], sem, add=True)
            s1.wait()
            s2.wait()
            return carry

        lax.fori_loop(0, nwin, window, 0)
        plsc.subcore_barrier()

        for c in range(rows_pt // k1):
            r = r0 + c * k1
            pltpu.sync_copy(acc_sp.at[pl.ds(r, k1)], hrows)
            pltpu.sync_copy(hrows, acc_out.at[cid, pl.ds(r, k1)])
        for c in range(rows_pt * 16 // (k1 * 8)):
            f0 = r0 * 16 + c * k1 * 8
            pltpu.sync_copy(den_sp.at[pl.ds(f0, k1 * 8)], exr)
            pltpu.sync_copy(exr, den_out.at[cid, pl.ds(f0, k1 * 8)])

    return edge1


def _mid_body(acc_ref, den_ref, rexp_ref, b1_ref, g_ref, bt_ref, mu_ref,
              va_ref, w2_ref, a2s_ref, a2d_ref, h2_ref, as2_ref, ad2_ref):
    a = acc_ref[0] + acc_ref[1]
    d = den_ref[0] + den_ref[1]
    dexp = jnp.dot(d, rexp_ref[...], preferred_element_type=F32)
    h1 = a / (dexp + 1e-16) + b1_ref[...]
    scale = g_ref[...] * lax.rsqrt(va_ref[...] + 1e-5)
    h1 = (h1 - mu_ref[...]) * scale + bt_ref[...]
    h1 = jnp.where(h1 > 0, h1, jnp.exp(h1) - 1.0)
    h2 = jnp.dot(h1, w2_ref[...], preferred_element_type=F32)
    h2_ref[...] = h2
    as2_ref[...] = jnp.dot(h2, a2s_ref[...], preferred_element_type=F32)
    ad2_ref[...] = jnp.dot(h2, a2d_ref[...], preferred_element_type=F32)


def _mid(acc1, den1, Rexp, b1r, g, bt, mu, va, W2, a2s, a2d, n, bn):
    hid = W2.shape[1]
    grid = (n // bn,)
    return pl.pallas_call(
        _mid_body,
        grid=grid,
        in_specs=[
            pl.BlockSpec((2, bn, 128), lambda i: (0, i, 0)),
            pl.BlockSpec((2, bn, 16), lambda i: (0, i, 0)),
            pl.BlockSpec((16, 128), lambda i: (0, 0)),
            pl.BlockSpec((1, 128), lambda i: (0, 0)),
            pl.BlockSpec((1, 128), lambda i: (0, 0)),
            pl.BlockSpec((1, 128), lambda i: (0, 0)),
            pl.BlockSpec((1, 128), lambda i: (0, 0)),
            pl.BlockSpec((1, 128), lambda i: (0, 0)),
            pl.BlockSpec((128, hid), lambda i: (0, 0)),
            pl.BlockSpec((hid, 128), lambda i: (0, 0)),
            pl.BlockSpec((hid, 128), lambda i: (0, 0)),
        ],
        out_specs=[
            pl.BlockSpec((bn, hid), lambda i: (i, 0)),
            pl.BlockSpec((bn, 128), lambda i: (i, 0)),
            pl.BlockSpec((bn, 128), lambda i: (i, 0)),
        ],
        out_shape=[
            jax.ShapeDtypeStruct((n, hid), F32),
            jax.ShapeDtypeStruct((n, 128), F32),
            jax.ShapeDtypeStruct((n, 128), F32),
        ],
    )(acc1, den1, Rexp, b1r, g, bt, mu, va, W2, a2s, a2d)


def _make_edge2(npad, e):
    k2 = 512
    nwin_total = e // k2
    nbase = nwin_total // NWORK
    nrem = nwin_total % NWORK
    rows_pt = npad // NS
    mesh = plsc.VectorSubcoreMesh(core_axis_name="c", subcore_axis_name="s")

    @functools.partial(
        pl.kernel,
        mesh=mesh,
        out_type=(
            jax.ShapeDtypeStruct((NC, npad * 16), F32),
            jax.ShapeDtypeStruct((NC, npad), F32),
        ),
        scratch_types=[
            pltpu.VMEM_SHARED((npad * 16,), F32),
            pltpu.VMEM_SHARED((npad,), F32),
            pltpu.VMEM((k2,), F32),
            pltpu.VMEM((k2,), F32),
            pltpu.VMEM((k2,), F32),
            pltpu.VMEM((k2,), jnp.int32),
            pltpu.VMEM((k2,), jnp.int32),
            pltpu.VMEM((k2 * 16,), jnp.int32),
            pltpu.VMEM((k2 * 16,), jnp.int32),
            pltpu.VMEM((k2 * 16,), F32),
            pltpu.VMEM((k2 * 16,), F32),
            pltpu.SemaphoreType.DMA,
        ],
    )
    def edge2(src_hbm, dst_hbm, h2f_hbm, as_hbm, ad_hbm,
              acc_out, den_out,
              acc_sp, den_sp, asr2, adr2, exs, sidx, didx, idxg, idxm,
              h2r, msg, sem):
        cid = lax.axis_index("c")
        sid = lax.axis_index("s")
        wid = sid * NC + cid
        r0 = sid * rows_pt
        lane = lax.iota(jnp.int32, 16)
        fpt = rows_pt * 16
        chunk = 2048
        nch = fpt // chunk

        def zb(i, c):
            msg[pl.ds(i * 16, 16)] = jnp.zeros((16,), F32)
            return c

        lax.fori_loop(0, chunk // 16, zb, 0)

        def zex(i, c):
            exs[pl.ds(i * 16, 16)] = jnp.zeros((16,), F32)
            return c

        lax.fori_loop(0, k2 // 16, zex, 0)
        for c in range(nch):
            pltpu.sync_copy(msg.at[pl.ds(0, chunk)],
                            acc_sp.at[pl.ds(r0 * 16 + c * chunk, chunk)])
        for c in range(rows_pt // k2 if rows_pt >= k2 else 0):
            pltpu.sync_copy(exs, den_sp.at[pl.ds(r0 + c * k2, k2)])
        for c in range(rows_pt % k2 // 128):
            pltpu.sync_copy(
                exs.at[pl.ds(0, 128)],
                den_sp.at[pl.ds(r0 + (rows_pt // k2) * k2 + c * 128, 128)])
        plsc.subcore_barrier()

        nwin = nbase + jnp.where(wid < nrem, 1, 0)

        def window(w, carry):
            off = (wid + NWORK * w) * k2
            c1 = pltpu.async_copy(src_hbm.at[pl.ds(off, k2)], sidx, sem)
            c2 = pltpu.async_copy(dst_hbm.at[pl.ds(off, k2)], didx, sem)
            c1.wait()
            c2.wait()
            ga = pltpu.async_copy(as_hbm.at[sidx], asr2, sem)
            gb = pltpu.async_copy(ad_hbm.at[didx], adr2, sem)

            def idx_group(j, c):
                svec = sidx[pl.ds(j * 16, 16)]
                dvec = didx[pl.ds(j * 16, 16)]
                for t in range(16):
                    k = j * 16 + t
                    idxg[pl.ds(k * 16, 16)] = svec[t] * 16 + lane
                    idxm[pl.ds(k * 16, 16)] = dvec[t] * 16 + lane
                return c

            lax.fori_loop(0, k2 // 16, idx_group, 0)
            gh = pltpu.async_copy(h2f_hbm.at[idxg], h2r, sem)
            ga.wait()
            gb.wait()
            gh.wait()

            def per_group(j, c):
                sv = asr2[pl.ds(j * 16, 16)]
                dv = adr2[pl.ds(j * 16, 16)]
                v = sv + dv
                v = jnp.where(v > 0, v, v * 0.2)
                ex = jnp.exp(v)
                exs[pl.ds(j * 16, 16)] = ex
                for t in range(16):
                    k = j * 16 + t
                    msg[pl.ds(k * 16, 16)] = h2r[pl.ds(k * 16, 16)] * ex[t]
                return c

            lax.fori_loop(0, k2 // 16, per_group, 0)
            s1 = pltpu.async_copy(msg, acc_sp.at[idxm], sem, add=True)
            s2 = pltpu.async_copy(exs, den_sp.at[didx], sem, add=True)
            s1.wait()
            s2.wait()
            return carry

        lax.fori_loop(0, nwin, window, 0)
        plsc.subcore_barrier()
        for c in range(nch):
            f0 = r0 * 16 + c * chunk
            pltpu.sync_copy(acc_sp.at[pl.ds(f0, chunk)],
                            msg.at[pl.ds(0, chunk)])
            pltpu.sync_copy(msg.at[pl.ds(0, chunk)],
                            acc_out.at[cid, pl.ds(f0, chunk)])
        for c in range(rows_pt // 128):
            f0 = r0 + c * 128
            pltpu.sync_copy(den_sp.at[pl.ds(f0, 128)], exs.at[pl.ds(0, 128)])
            pltpu.sync_copy(exs.at[pl.ds(0, 128)],
                            den_out.at[cid, pl.ds(f0, 128)])

    return edge2


def _head_body(acc_ref, den_ref, ones_ref, b2_ref, wc_ref, bc_ref, ws_ref,
               bs_ref, wh_ref, bh_ref, we_ref, be_ref,
               main_ref, sim_ref, hom_ref, ent_ref):
    num = acc_ref[0] + acc_ref[1]
    d = den_ref[0] + den_ref[1]
    den = jnp.dot(d, ones_ref[...], preferred_element_type=F32)
    h = num / (den + 1e-16) + b2_ref[...]
    mo = jnp.dot(h, wc_ref[...], preferred_element_type=F32) + bc_ref[...]
    mo = mo - jnp.max(mo, axis=1, keepdims=True)
    main_ref[...] = mo - jnp.log(jnp.sum(jnp.exp(mo), axis=1, keepdims=True))
    so = jnp.dot(h, ws_ref[...], preferred_element_type=F32) + bs_ref[...]
    so = jnp.exp(so - jnp.max(so, axis=1, keepdims=True))
    sim_ref[...] = so / jnp.sum(so, axis=1, keepdims=True)
    hom_ref[...] = jax.nn.sigmoid(
        jnp.dot(h, wh_ref[...], preferred_element_type=F32) + bh_ref[...])
    ent_ref[...] = jax.nn.sigmoid(
        jnp.dot(h, we_ref[...], preferred_element_type=F32) + be_ref[...])


def _heads(acc2, den2, ones16, b2r, Wc, bcr, Ws, bsr, Wh, bhr, We, ber,
           n, bn):
    out_c = Wc.shape[1]
    ncls = Ws.shape[1]
    grid = (n // bn,)

    def cst(shp):
        return pl.BlockSpec(shp, lambda *_: tuple(0 for _ in shp))

    return pl.pallas_call(
        _head_body,
        grid=grid,
        in_specs=[
            pl.BlockSpec((2, bn, 16), lambda i: (0, i, 0)),
            pl.BlockSpec((2, bn, 1), lambda i: (0, i, 0)),
            cst((1, 16)), cst((1, 16)),
            cst((16, out_c)), cst((1, out_c)),
            cst((16, ncls)), cst((1, ncls)),
            cst((16, 1)), cst((1, 1)),
            cst((16, 1)), cst((1, 1)),
        ],
        out_specs=[
            pl.BlockSpec((bn, out_c), lambda i: (i, 0)),
            pl.BlockSpec((bn, ncls), lambda i: (i, 0)),
            pl.BlockSpec((bn, 1), lambda i: (i, 0)),
            pl.BlockSpec((bn, 1), lambda i: (i, 0)),
        ],
        out_shape=[
            jax.ShapeDtypeStruct((n, out_c), F32),
            jax.ShapeDtypeStruct((n, ncls), F32),
            jax.ShapeDtypeStruct((n, 1), F32),
            jax.ShapeDtypeStruct((n, 1), F32),
        ],
    )(acc2, den2, ones16, b2r, Wc, bcr, Ws, bsr, Wh, bhr, We, ber)


def kernel(x, edge_index, W1, att_src1, att_dst1, b1, bn_gamma, bn_beta,
           bn_mean, bn_var, W2, att_src2, att_dst2, b2, Wc, bc, Ws, bs,
           Wh, bh, We, be):
    n, din = x.shape
    e = edge_index.shape[1]
    heads, hid = att_src1.shape
    dh = heads * hid

    src = edge_index[0]
    dst = edge_index[1]

    A1s = jnp.zeros((dh, 16), F32)
    A1d = jnp.zeros((dh, 16), F32)
    for h in range(heads):
        A1s = A1s.at[h * hid:(h + 1) * hid, h].set(att_src1[h])
        A1d = A1d.at[h * hid:(h + 1) * hid, h].set(att_dst1[h])

    rexp = np.zeros((16, dh), np.float32)
    for h in range(heads):
        rexp[h, h * hid:(h + 1) * hid] = 1.0
    Rexp = jnp.asarray(rexp)

    bn = 1000
    npad = 10240 if n == 10000 else ((n + NS * 128 - 1) // (NS * 128)) * NS * 128
    h1, as1, ad1 = _dense1(x, W1, A1s, A1d, bn)

    edge1 = _make_edge1(npad, n, e)
    alcat = jnp.concatenate([as1.reshape(-1), ad1.reshape(-1)])
    acc1, den1f = edge1(src, dst, h1, alcat)
    den1 = den1f.reshape(NC, npad, 16)

    h2, as2, ad2 = _mid(
        acc1, den1, Rexp, b1.reshape(1, dh), bn_gamma.reshape(1, dh),
        bn_beta.reshape(1, dh), bn_mean.reshape(1, dh), bn_var.reshape(1, dh),
        W2, jnp.tile(att_src2.reshape(hid, 1), (1, 128)),
        jnp.tile(att_dst2.reshape(hid, 1), (1, 128)), n, bn)

    edge2 = _make_edge2(npad, e)
    acc2f, den2f = edge2(src, dst, h2.reshape(-1), as2[:, 0], ad2[:, 0])
    acc2 = acc2f.reshape(NC, npad, 16)
    den2 = den2f.reshape(NC, npad, 1)

    main, sim, hom, ent = _heads(
        acc2, den2, jnp.ones((1, 16), F32), b2.reshape(1, hid),
        Wc, bc.reshape(1, -1),
        Ws, bs.reshape(1, -1), Wh, bh.reshape(1, 1), We, be.reshape(1, 1),
        n, bn)
    return main, sim, hom[:, 0], ent[:, 0]


# speedup vs baseline: 48.2020x; 1.0131x over previous
import functools
import numpy as np
import jax
import jax.numpy as jnp
from jax import lax
from jax.experimental import pallas as pl
from jax.experimental.pallas import tpu as pltpu
from jax.experimental.pallas import tpu_sc as plsc

NC = 2
NS = 16
NWORK = NC * NS

F32 = jnp.float32

_GDN = jax.lax.GatherDimensionNumbers(
    offset_dims=(), collapsed_slice_dims=(0,), start_index_map=(0,))


def _vtake(vec, idx):
    return lax.gather(vec, idx[:, None], _GDN, (1,),
                      mode=lax.GatherScatterMode.PROMISE_IN_BOUNDS)


def _dense1_body(x_ref, w1_ref, a1s_ref, a1d_ref, h_ref, as_ref, ad_ref):
    h = jnp.dot(x_ref[...], w1_ref[...], preferred_element_type=F32)
    h_ref[...] = h
    as_ref[...] = jnp.dot(h, a1s_ref[...], preferred_element_type=F32)
    ad_ref[...] = jnp.dot(h, a1d_ref[...], preferred_element_type=F32)


def _dense1(x, W1, A1s, A1d, bn):
    n, din = x.shape
    dh = W1.shape[1]
    grid = (n // bn,)
    return pl.pallas_call(
        _dense1_body,
        grid=grid,
        in_specs=[
            pl.BlockSpec((bn, din), lambda i: (i, 0)),
            pl.BlockSpec((din, dh), lambda i: (0, 0)),
            pl.BlockSpec((dh, 16), lambda i: (0, 0)),
            pl.BlockSpec((dh, 16), lambda i: (0, 0)),
        ],
        out_specs=[
            pl.BlockSpec((bn, dh), lambda i: (i, 0)),
            pl.BlockSpec((bn, 16), lambda i: (i, 0)),
            pl.BlockSpec((bn, 16), lambda i: (i, 0)),
        ],
        out_shape=[
            jax.ShapeDtypeStruct((n, dh), F32),
            jax.ShapeDtypeStruct((n, 16), F32),
            jax.ShapeDtypeStruct((n, 16), F32),
        ],
    )(x, W1, A1s, A1d)


def _make_edge1(npad, n, e):
    k1 = 160
    n16 = n * 16
    nwin_total = e // k1
    nbase = nwin_total // NWORK
    nrem = nwin_total % NWORK
    rows_pt = npad // NS
    mesh = plsc.VectorSubcoreMesh(core_axis_name="c", subcore_axis_name="s")

    @functools.partial(
        pl.kernel,
        mesh=mesh,
        out_type=(
            jax.ShapeDtypeStruct((NC, npad, 128), F32),
            jax.ShapeDtypeStruct((NC, npad * 8), F32),
        ),
        scratch_types=[
            pltpu.VMEM_SHARED((npad, 128), F32),
            pltpu.VMEM_SHARED((npad * 8,), F32),
            pltpu.VMEM((k1,), jnp.int32),
            pltpu.VMEM((k1,), jnp.int32),
            pltpu.VMEM((k1 * 16,), jnp.int32),
            pltpu.VMEM((k1 * 8,), jnp.int32),
            pltpu.VMEM((k1 * 16,), F32),
            pltpu.VMEM((k1 * 8,), F32),
            pltpu.VMEM((k1, 128), F32),
            pltpu.SemaphoreType.DMA,
        ],
    )
    def edge1(src_hbm, dst_hbm, h_hbm, alcat_hbm,
              acc_out, den_out,
              acc_sp, den_sp, sidx, didx, idx1, idx2, asr, exr,
              hrows, sem):
        cid = lax.axis_index("c")
        sid = lax.axis_index("s")
        wid = sid * NC + cid
        r0 = sid * rows_pt
        lane = lax.iota(jnp.int32, 16)

        def zrow(r, c):
            for hh in range(8):
                hrows[r, pl.ds(hh * 16, 16)] = jnp.zeros((16,), F32)
            return c

        lax.fori_loop(0, k1, zrow, 0)

        def zex(i, c):
            exr[pl.ds(i * 16, 16)] = jnp.zeros((16,), F32)
            return c

        lax.fori_loop(0, k1 * 8 // 16, zex, 0)
        for c in range(rows_pt // k1):
            r = r0 + c * k1
            pltpu.sync_copy(hrows, acc_sp.at[pl.ds(r, k1)])
        for c in range(rows_pt * 8 // (k1 * 8)):
            f0 = r0 * 8 + c * k1 * 8
            pltpu.sync_copy(exr, den_sp.at[pl.ds(f0, k1 * 8)])
        plsc.subcore_barrier()

        nwin = nbase + jnp.where(wid < nrem, 1, 0)

        def window(w, carry):
            off = (wid + NWORK * w) * k1
            c1 = pltpu.async_copy(src_hbm.at[pl.ds(off, k1)], sidx, sem)
            c2 = pltpu.async_copy(dst_hbm.at[pl.ds(off, k1)], didx, sem)
            c1.wait()
            c2.wait()
            gh = pltpu.async_copy(h_hbm.at[sidx], hrows, sem)
            mlo = lane < 8
            la7 = lane & 7

            def idx_group(j, c):
                svec = sidx[pl.ds(j * 16, 16)]
                dvec = didx[pl.ds(j * 16, 16)]
                for t in range(16):
                    k = j * 16 + t
                    idx1[pl.ds(k * 16, 16)] = (
                        jnp.where(mlo, svec[t] * 16, n16 + dvec[t] * 16)
                        + la7)
                    if t % 2 == 0:
                        p = j * 8 + t // 2
                        idx2[pl.ds(p * 16, 16)] = (
                            jnp.where(mlo, dvec[t] * 8, dvec[t + 1] * 8)
                            + la7)
                return c

            lax.fori_loop(0, k1 // 16, idx_group, 0)
            ga = pltpu.async_copy(alcat_hbm.at[idx1], asr, sem)
            ga.wait()
            gh.wait()

            def per_pair(p, c):
                e0 = 2 * p
                e1 = 2 * p + 1
                g0 = asr[pl.ds(e0 * 16, 16)]
                v0 = _vtake(g0, la7) + _vtake(g0, la7 + 8)
                v0 = jnp.where(v0 > 0, v0, v0 * 0.2)
                ex0 = jnp.exp(v0)
                g1 = asr[pl.ds(e1 * 16, 16)]
                v1 = _vtake(g1, la7) + _vtake(g1, la7 + 8)
                v1 = jnp.where(v1 > 0, v1, v1 * 0.2)
                ex1 = jnp.exp(v1)
                exr[pl.ds(p * 16, 16)] = jnp.where(mlo, ex0, ex1)
                for hh in range(8):
                    hrows[e0, pl.ds(hh * 16, 16)] = (
                        hrows[e0, pl.ds(hh * 16, 16)] * ex0[hh])
                    hrows[e1, pl.ds(hh * 16, 16)] = (
                        hrows[e1, pl.ds(hh * 16, 16)] * ex1[hh])
                return c

            lax.fori_loop(0, k1 // 2, per_pair, 0)
            s1 = pltpu.async_copy(exr, den_sp.at[idx2], sem, add=True)
            s2 = pltpu.async_copy(hrows, acc_sp.at[didx], sem, add=True)
            s1.wait()
            s2.wait()
            return carry

        lax.fori_loop(0, nwin, window, 0)
        plsc.subcore_barrier()

        for c in range(rows_pt // k1):
            r = r0 + c * k1
            pltpu.sync_copy(acc_sp.at[pl.ds(r, k1)], hrows)
            pltpu.sync_copy(hrows, acc_out.at[cid, pl.ds(r, k1)])
        for c in range(rows_pt * 8 // (k1 * 8)):
            f0 = r0 * 8 + c * k1 * 8
            pltpu.sync_copy(den_sp.at[pl.ds(f0, k1 * 8)], exr)
            pltpu.sync_copy(exr, den_out.at[cid, pl.ds(f0, k1 * 8)])

    return edge1


def _mid_body(acc_ref, den_ref, rexp_ref, b1_ref, g_ref, bt_ref, mu_ref,
              va_ref, w2_ref, a2s_ref, a2d_ref, h2_ref, as2_ref, ad2_ref):
    a = acc_ref[0] + acc_ref[1]
    d = den_ref[0] + den_ref[1]
    dexp = jnp.dot(d, rexp_ref[...], preferred_element_type=F32)
    h1 = a / (dexp + 1e-16) + b1_ref[...]
    scale = g_ref[...] * lax.rsqrt(va_ref[...] + 1e-5)
    h1 = (h1 - mu_ref[...]) * scale + bt_ref[...]
    h1 = jnp.where(h1 > 0, h1, jnp.exp(h1) - 1.0)
    h2 = jnp.dot(h1, w2_ref[...], preferred_element_type=F32)
    h2_ref[...] = h2
    as2_ref[...] = jnp.dot(h2, a2s_ref[...], preferred_element_type=F32)
    ad2_ref[...] = jnp.dot(h2, a2d_ref[...], preferred_element_type=F32)


def _mid(acc1, den1, Rexp, b1r, g, bt, mu, va, W2, a2s, a2d, n, bn):
    hid = W2.shape[1]
    grid = (n // bn,)
    return pl.pallas_call(
        _mid_body,
        grid=grid,
        in_specs=[
            pl.BlockSpec((2, bn, 128), lambda i: (0, i, 0)),
            pl.BlockSpec((2, bn, 8), lambda i: (0, i, 0)),
            pl.BlockSpec((8, 128), lambda i: (0, 0)),
            pl.BlockSpec((1, 128), lambda i: (0, 0)),
            pl.BlockSpec((1, 128), lambda i: (0, 0)),
            pl.BlockSpec((1, 128), lambda i: (0, 0)),
            pl.BlockSpec((1, 128), lambda i: (0, 0)),
            pl.BlockSpec((1, 128), lambda i: (0, 0)),
            pl.BlockSpec((128, hid), lambda i: (0, 0)),
            pl.BlockSpec((hid, 128), lambda i: (0, 0)),
            pl.BlockSpec((hid, 128), lambda i: (0, 0)),
        ],
        out_specs=[
            pl.BlockSpec((bn, hid), lambda i: (i, 0)),
            pl.BlockSpec((bn, 128), lambda i: (i, 0)),
            pl.BlockSpec((bn, 128), lambda i: (i, 0)),
        ],
        out_shape=[
            jax.ShapeDtypeStruct((n, hid), F32),
            jax.ShapeDtypeStruct((n, 128), F32),
            jax.ShapeDtypeStruct((n, 128), F32),
        ],
    )(acc1, den1, Rexp, b1r, g, bt, mu, va, W2, a2s, a2d)


def _make_edge2(npad, n, e):
    k2 = 512
    npt16 = n * 16 // NS
    nwin_total = e // k2
    nbase = nwin_total // NWORK
    nrem = nwin_total % NWORK
    rows_pt = npad // NS
    mesh = plsc.VectorSubcoreMesh(core_axis_name="c", subcore_axis_name="s")

    @functools.partial(
        pl.kernel,
        mesh=mesh,
        out_type=(
            jax.ShapeDtypeStruct((NC, npad * 16), F32),
            jax.ShapeDtypeStruct((NC, npad), F32),
        ),
        scratch_types=[
            pltpu.VMEM_SHARED((npad * 16,), F32),
            pltpu.VMEM_SHARED((npad,), F32),
            pltpu.VMEM((k2,), F32),
            pltpu.VMEM((k2,), F32),
            pltpu.VMEM((k2,), F32),
            pltpu.VMEM((k2,), jnp.int32),
            pltpu.VMEM((k2,), jnp.int32),
            pltpu.VMEM((k2 * 16,), jnp.int32),
            pltpu.VMEM((k2 * 16,), jnp.int32),
            pltpu.VMEM((k2 * 16,), F32),
            pltpu.VMEM((k2 * 16,), F32),
            pltpu.SemaphoreType.DMA,
        ],
    )
    def edge2(src_hbm, dst_hbm, h2f_hbm, as_hbm, ad_hbm,
              acc_out, den_out,
              acc_sp, den_sp, asr2, adr2, exs, sidx, didx, idxg, idxm,
              h2r, msg, sem):
        cid = lax.axis_index("c")
        sid = lax.axis_index("s")
        wid = sid * NC + cid
        r0 = sid * rows_pt
        lane = lax.iota(jnp.int32, 16)
        fpt = rows_pt * 16
        chunk = 2048
        nch = fpt // chunk

        def zb(i, c):
            msg[pl.ds(i * 16, 16)] = jnp.zeros((16,), F32)
            return c

        lax.fori_loop(0, chunk // 16, zb, 0)

        def zex(i, c):
            exs[pl.ds(i * 16, 16)] = jnp.zeros((16,), F32)
            return c

        lax.fori_loop(0, k2 // 16, zex, 0)
        for c in range(nch):
            pltpu.sync_copy(msg.at[pl.ds(0, chunk)],
                            acc_sp.at[pl.ds(r0 * 16 + c * chunk, chunk)])
        for c in range(rows_pt // k2 if rows_pt >= k2 else 0):
            pltpu.sync_copy(exs, den_sp.at[pl.ds(r0 + c * k2, k2)])
        for c in range(rows_pt % k2 // 128):
            pltpu.sync_copy(
                exs.at[pl.ds(0, 128)],
                den_sp.at[pl.ds(r0 + (rows_pt // k2) * k2 + c * 128, 128)])
        plsc.subcore_barrier()

        nwin = nbase + jnp.where(wid < nrem, 1, 0)

        def window(w, carry):
            off = (wid + NWORK * w) * k2
            c1 = pltpu.async_copy(src_hbm.at[pl.ds(off, k2)], sidx, sem)
            c2 = pltpu.async_copy(dst_hbm.at[pl.ds(off, k2)], didx, sem)
            c1.wait()
            c2.wait()
            ga = pltpu.async_copy(as_hbm.at[sidx], asr2, sem)
            gb = pltpu.async_copy(ad_hbm.at[didx], adr2, sem)

            def idx_group(j, c):
                svec = sidx[pl.ds(j * 16, 16)]
                dvec = didx[pl.ds(j * 16, 16)]
                for t in range(16):
                    k = j * 16 + t
                    idxg[pl.ds(k * 16, 16)] = svec[t] * 16 + lane
                    idxm[pl.ds(k * 16, 16)] = dvec[t] * 16 + lane
                return c

            lax.fori_loop(0, k2 // 16, idx_group, 0)
            gh = pltpu.async_copy(h2f_hbm.at[idxg], h2r, sem)
            ga.wait()
            gb.wait()
            gh.wait()

            def per_group(j, c):
                sv = asr2[pl.ds(j * 16, 16)]
                dv = adr2[pl.ds(j * 16, 16)]
                v = sv + dv
                v = jnp.where(v > 0, v, v * 0.2)
                ex = jnp.exp(v)
                exs[pl.ds(j * 16, 16)] = ex
                for t in range(16):
                    k = j * 16 + t
                    msg[pl.ds(k * 16, 16)] = h2r[pl.ds(k * 16, 16)] * ex[t]
                return c

            lax.fori_loop(0, k2 // 16, per_group, 0)
            s1 = pltpu.async_copy(msg, acc_sp.at[idxm], sem, add=True)
            s2 = pltpu.async_copy(exs, den_sp.at[didx], sem, add=True)
            s1.wait()
            s2.wait()
            return carry

        lax.fori_loop(0, nwin, window, 0)
        plsc.subcore_barrier()
        for c in range(nch):
            f0 = r0 * 16 + c * chunk
            pltpu.sync_copy(acc_sp.at[pl.ds(f0, chunk)],
                            msg.at[pl.ds(0, chunk)])
            pltpu.sync_copy(msg.at[pl.ds(0, chunk)],
                            acc_out.at[cid, pl.ds(f0, chunk)])
        for c in range(rows_pt // 128):
            f0 = r0 + c * 128
            pltpu.sync_copy(den_sp.at[pl.ds(f0, 128)], exs.at[pl.ds(0, 128)])
            pltpu.sync_copy(exs.at[pl.ds(0, 128)],
                            den_out.at[cid, pl.ds(f0, 128)])

    return edge2


def _head_body(acc_ref, den_ref, ones_ref, b2_ref, wc_ref, bc_ref, ws_ref,
               bs_ref, wh_ref, bh_ref, we_ref, be_ref,
               main_ref, sim_ref, hom_ref, ent_ref):
    num = acc_ref[0] + acc_ref[1]
    d = den_ref[0] + den_ref[1]
    den = jnp.dot(d, ones_ref[...], preferred_element_type=F32)
    h = num / (den + 1e-16) + b2_ref[...]
    mo = jnp.dot(h, wc_ref[...], preferred_element_type=F32) + bc_ref[...]
    mo = mo - jnp.max(mo, axis=1, keepdims=True)
    main_ref[...] = mo - jnp.log(jnp.sum(jnp.exp(mo), axis=1, keepdims=True))
    so = jnp.dot(h, ws_ref[...], preferred_element_type=F32) + bs_ref[...]
    so = jnp.exp(so - jnp.max(so, axis=1, keepdims=True))
    sim_ref[...] = so / jnp.sum(so, axis=1, keepdims=True)
    hom_ref[...] = jax.nn.sigmoid(
        jnp.dot(h, wh_ref[...], preferred_element_type=F32) + bh_ref[...])
    ent_ref[...] = jax.nn.sigmoid(
        jnp.dot(h, we_ref[...], preferred_element_type=F32) + be_ref[...])


def _heads(acc2, den2, ones16, b2r, Wc, bcr, Ws, bsr, Wh, bhr, We, ber,
           n, bn):
    out_c = Wc.shape[1]
    ncls = Ws.shape[1]
    grid = (n // bn,)

    def cst(shp):
        return pl.BlockSpec(shp, lambda *_: tuple(0 for _ in shp))

    return pl.pallas_call(
        _head_body,
        grid=grid,
        in_specs=[
            pl.BlockSpec((2, bn, 16), lambda i: (0, i, 0)),
            pl.BlockSpec((2, bn, 1), lambda i: (0, i, 0)),
            cst((1, 16)), cst((1, 16)),
            cst((16, out_c)), cst((1, out_c)),
            cst((16, ncls)), cst((1, ncls)),
            cst((16, 1)), cst((1, 1)),
            cst((16, 1)), cst((1, 1)),
        ],
        out_specs=[
            pl.BlockSpec((bn, out_c), lambda i: (i, 0)),
            pl.BlockSpec((bn, ncls), lambda i: (i, 0)),
            pl.BlockSpec((bn, 1), lambda i: (i, 0)),
            pl.BlockSpec((bn, 1), lambda i: (i, 0)),
        ],
        out_shape=[
            jax.ShapeDtypeStruct((n, out_c), F32),
            jax.ShapeDtypeStruct((n, ncls), F32),
            jax.ShapeDtypeStruct((n, 1), F32),
            jax.ShapeDtypeStruct((n, 1), F32),
        ],
    )(acc2, den2, ones16, b2r, Wc, bcr, Ws, bsr, Wh, bhr, We, ber)


def kernel(x, edge_index, W1, att_src1, att_dst1, b1, bn_gamma, bn_beta,
           bn_mean, bn_var, W2, att_src2, att_dst2, b2, Wc, bc, Ws, bs,
           Wh, bh, We, be):
    n, din = x.shape
    e = edge_index.shape[1]
    heads, hid = att_src1.shape
    dh = heads * hid

    src = edge_index[0]
    dst = edge_index[1]

    A1s = jnp.zeros((dh, 16), F32)
    A1d = jnp.zeros((dh, 16), F32)
    for h in range(heads):
        A1s = A1s.at[h * hid:(h + 1) * hid, h].set(att_src1[h])
        A1d = A1d.at[h * hid:(h + 1) * hid, h].set(att_dst1[h])

    rexp = np.zeros((8, dh), np.float32)
    for h in range(heads):
        rexp[h, h * hid:(h + 1) * hid] = 1.0
    Rexp = jnp.asarray(rexp)

    bn = 1000
    npad = 10240 if n == 10000 else ((n + NS * 128 - 1) // (NS * 128)) * NS * 128
    h1, as1, ad1 = _dense1(x, W1, A1s, A1d, bn)

    edge1 = _make_edge1(npad, n, e)
    alcat = jnp.concatenate([as1.reshape(-1), ad1.reshape(-1)])
    acc1, den1f = edge1(src, dst, h1, alcat)
    den1 = den1f.reshape(NC, npad, 8)

    h2, as2, ad2 = _mid(
        acc1, den1, Rexp, b1.reshape(1, dh), bn_gamma.reshape(1, dh),
        bn_beta.reshape(1, dh), bn_mean.reshape(1, dh), bn_var.reshape(1, dh),
        W2, jnp.tile(att_src2.reshape(hid, 1), (1, 128)),
        jnp.tile(att_dst2.reshape(hid, 1), (1, 128)), n, bn)

    edge2 = _make_edge2(npad, n, e)
    acc2f, den2f = edge2(src, dst, h2.reshape(-1), as2[:, 0], ad2[:, 0])
    acc2 = acc2f.reshape(NC, npad, 16)
    den2 = den2f.reshape(NC, npad, 1)

    main, sim, hom, ent = _heads(
        acc2, den2, jnp.ones((1, 16), F32), b2.reshape(1, hid),
        Wc, bc.reshape(1, -1),
        Ws, bs.reshape(1, -1), Wh, bh.reshape(1, 1), We, be.reshape(1, 1),
        n, bn)
    return main, sim, hom[:, 0], ent[:, 0]


# speedup vs baseline: 51.2526x; 1.0633x over previous
import functools
import numpy as np
import jax
import jax.numpy as jnp
from jax import lax
from jax.experimental import pallas as pl
from jax.experimental.pallas import tpu as pltpu
from jax.experimental.pallas import tpu_sc as plsc

NC = 2
NS = 16
NWORK = NC * NS

F32 = jnp.float32

_GDN = jax.lax.GatherDimensionNumbers(
    offset_dims=(), collapsed_slice_dims=(0,), start_index_map=(0,))


def _vtake(vec, idx):
    return lax.gather(vec, idx[:, None], _GDN, (1,),
                      mode=lax.GatherScatterMode.PROMISE_IN_BOUNDS)


def _dense1_body(x_ref, w1_ref, a1s_ref, a1d_ref, h_ref, as_ref, ad_ref):
    h = jnp.dot(x_ref[...], w1_ref[...], preferred_element_type=F32)
    h_ref[...] = h
    as_ref[...] = jnp.dot(h, a1s_ref[...], preferred_element_type=F32)
    ad_ref[...] = jnp.dot(h, a1d_ref[...], preferred_element_type=F32)


def _dense1(x, W1, A1s, A1d, bn):
    n, din = x.shape
    dh = W1.shape[1]
    grid = (n // bn,)
    return pl.pallas_call(
        _dense1_body,
        grid=grid,
        in_specs=[
            pl.BlockSpec((bn, din), lambda i: (i, 0)),
            pl.BlockSpec((din, dh), lambda i: (0, 0)),
            pl.BlockSpec((dh, 16), lambda i: (0, 0)),
            pl.BlockSpec((dh, 16), lambda i: (0, 0)),
        ],
        out_specs=[
            pl.BlockSpec((bn, dh), lambda i: (i, 0)),
            pl.BlockSpec((bn, 16), lambda i: (i, 0)),
            pl.BlockSpec((bn, 16), lambda i: (i, 0)),
        ],
        out_shape=[
            jax.ShapeDtypeStruct((n, dh), F32),
            jax.ShapeDtypeStruct((n, 16), F32),
            jax.ShapeDtypeStruct((n, 16), F32),
        ],
    )(x, W1, A1s, A1d)


def _make_edge1(npad, n, e):
    k1 = 160
    n16 = n * 16
    nwin_total = e // k1
    nbase = nwin_total // NWORK
    nrem = nwin_total % NWORK
    rows_pt = npad // NS
    mesh = plsc.VectorSubcoreMesh(core_axis_name="c", subcore_axis_name="s")

    @functools.partial(
        pl.kernel,
        mesh=mesh,
        out_type=(
            jax.ShapeDtypeStruct((NC, npad, 128), F32),
            jax.ShapeDtypeStruct((NC, npad * 8), F32),
        ),
        scratch_types=[
            pltpu.VMEM_SHARED((npad, 128), F32),
            pltpu.VMEM_SHARED((npad * 8,), F32),
            pltpu.VMEM((k1,), jnp.int32),
            pltpu.VMEM((k1 // 2,), jnp.int32),
            pltpu.VMEM((k1 // 2,), jnp.int32),
            pltpu.VMEM((k1 * 8,), jnp.int32),
            pltpu.VMEM((k1 * 8,), jnp.int32),
            pltpu.VMEM((k1 * 4,), jnp.int32),
            pltpu.VMEM((k1 * 4,), jnp.int32),
            pltpu.VMEM((k1 * 8,), F32),
            pltpu.VMEM((k1 * 8,), F32),
            pltpu.VMEM((k1 * 4,), F32),
            pltpu.VMEM((k1 * 4,), F32),
            pltpu.VMEM((k1 // 2, 128), F32),
            pltpu.VMEM((k1 // 2, 128), F32),
            pltpu.SemaphoreType.DMA,
            pltpu.SemaphoreType.DMA,
        ],
    )
    def edge1(src_hbm, dst_hbm, h_hbm, alcat_hbm,
              acc_out, den_out,
              acc_sp, den_sp, sidx, didxA, didxB, idx1A, idx1B, idx2A,
              idx2B, asrA, asrB, exrA, exrB, hrowsA, hrowsB, semg, sems):
        cid = lax.axis_index("c")
        sid = lax.axis_index("s")
        wid = sid * NC + cid
        r0 = sid * rows_pt
        lane = lax.iota(jnp.int32, 16)

        kh = k1 // 2
        mlo = lane < 8
        la7 = lane & 7

        def zrow(r, c):
            for hh in range(8):
                hrowsA[r, pl.ds(hh * 16, 16)] = jnp.zeros((16,), F32)
            return c

        lax.fori_loop(0, kh, zrow, 0)

        def zex(i, c):
            exrA[pl.ds(i * 16, 16)] = jnp.zeros((16,), F32)
            return c

        lax.fori_loop(0, k1 * 4 // 16, zex, 0)
        for c in range(rows_pt // kh):
            r = r0 + c * kh
            pltpu.sync_copy(hrowsA, acc_sp.at[pl.ds(r, kh)])
        for c in range(rows_pt * 8 // (k1 * 4)):
            f0 = r0 * 8 + c * k1 * 4
            pltpu.sync_copy(exrA, den_sp.at[pl.ds(f0, k1 * 4)])
        plsc.subcore_barrier()

        nwin = nbase + jnp.where(wid < nrem, 1, 0)

        def build_idx(h0, didxX, idx1X, idx2X):
            def idx_group(j, c):
                svec = sidx[pl.ds(h0 + j * 16, 16)]
                dvec = didxX[pl.ds(j * 16, 16)]
                for t in range(16):
                    k = j * 16 + t
                    idx1X[pl.ds(k * 16, 16)] = (
                        jnp.where(mlo, svec[t] * 16, n16 + dvec[t] * 16)
                        + la7)
                    if t % 2 == 0:
                        p = j * 8 + t // 2
                        idx2X[pl.ds(p * 16, 16)] = (
                            jnp.where(mlo, dvec[t] * 8, dvec[t + 1] * 8)
                            + la7)
                return c

            lax.fori_loop(0, kh // 16, idx_group, 0)

        def compute(asrX, exrX, hrowsX):
            def per_pair(p, c):
                e0 = 2 * p
                e1 = 2 * p + 1
                g0 = asrX[pl.ds(e0 * 16, 16)]
                v0 = _vtake(g0, la7) + _vtake(g0, la7 + 8)
                v0 = jnp.where(v0 > 0, v0, v0 * 0.2)
                ex0 = jnp.exp(v0)
                g1 = asrX[pl.ds(e1 * 16, 16)]
                v1 = _vtake(g1, la7) + _vtake(g1, la7 + 8)
                v1 = jnp.where(v1 > 0, v1, v1 * 0.2)
                ex1 = jnp.exp(v1)
                exrX[pl.ds(p * 16, 16)] = jnp.where(mlo, ex0, ex1)
                for hh in range(8):
                    hrowsX[e0, pl.ds(hh * 16, 16)] = (
                        hrowsX[e0, pl.ds(hh * 16, 16)] * ex0[hh])
                    hrowsX[e1, pl.ds(hh * 16, 16)] = (
                        hrowsX[e1, pl.ds(hh * 16, 16)] * ex1[hh])
                return c

            lax.fori_loop(0, kh // 2, per_pair, 0)

        def window(w, carry):
            off = (wid + NWORK * w) * k1
            c1 = pltpu.async_copy(src_hbm.at[pl.ds(off, k1)], sidx, semg)
            c2 = pltpu.async_copy(dst_hbm.at[pl.ds(off, kh)], didxA, semg)
            c3 = pltpu.async_copy(dst_hbm.at[pl.ds(off + kh, kh)], didxB,
                                  semg)
            c1.wait()
            c2.wait()
            c3.wait()
            ghA = pltpu.async_copy(h_hbm.at[sidx.at[pl.ds(0, kh)]], hrowsA,
                                   semg)
            ghB = pltpu.async_copy(h_hbm.at[sidx.at[pl.ds(kh, kh)]], hrowsB,
                                   semg)
            build_idx(0, didxA, idx1A, idx2A)
            gaA = pltpu.async_copy(alcat_hbm.at[idx1A], asrA, semg)
            build_idx(kh, didxB, idx1B, idx2B)
            gaB = pltpu.async_copy(alcat_hbm.at[idx1B], asrB, semg)
            gaA.wait()
            ghA.wait()
            compute(asrA, exrA, hrowsA)
            sA1 = pltpu.async_copy(exrA, den_sp.at[idx2A], sems, add=True)
            sA2 = pltpu.async_copy(hrowsA, acc_sp.at[didxA], sems, add=True)
            gaB.wait()
            ghB.wait()
            compute(asrB, exrB, hrowsB)
            sB1 = pltpu.async_copy(exrB, den_sp.at[idx2B], sems, add=True)
            sB2 = pltpu.async_copy(hrowsB, acc_sp.at[didxB], sems, add=True)
            sA1.wait()
            sA2.wait()
            sB1.wait()
            sB2.wait()
            return carry

        lax.fori_loop(0, nwin, window, 0)
        plsc.subcore_barrier()

        for c in range(rows_pt // kh):
            r = r0 + c * kh
            pltpu.sync_copy(acc_sp.at[pl.ds(r, kh)], hrowsA)
            pltpu.sync_copy(hrowsA, acc_out.at[cid, pl.ds(r, kh)])
        for c in range(rows_pt * 8 // (k1 * 4)):
            f0 = r0 * 8 + c * k1 * 4
            pltpu.sync_copy(den_sp.at[pl.ds(f0, k1 * 4)], exrA)
            pltpu.sync_copy(exrA, den_out.at[cid, pl.ds(f0, k1 * 4)])

    return edge1


def _mid_body(acc_ref, den_ref, rexp_ref, b1_ref, g_ref, bt_ref, mu_ref,
              va_ref, w2_ref, a2s_ref, a2d_ref, h2_ref, as2_ref, ad2_ref):
    a = acc_ref[0] + acc_ref[1]
    d = den_ref[0] + den_ref[1]
    dexp = jnp.dot(d, rexp_ref[...], preferred_element_type=F32)
    h1 = a / (dexp + 1e-16) + b1_ref[...]
    scale = g_ref[...] * lax.rsqrt(va_ref[...] + 1e-5)
    h1 = (h1 - mu_ref[...]) * scale + bt_ref[...]
    h1 = jnp.where(h1 > 0, h1, jnp.exp(h1) - 1.0)
    h2 = jnp.dot(h1, w2_ref[...], preferred_element_type=F32)
    h2_ref[...] = h2
    as2_ref[...] = jnp.dot(h2, a2s_ref[...], preferred_element_type=F32)
    ad2_ref[...] = jnp.dot(h2, a2d_ref[...], preferred_element_type=F32)


def _mid(acc1, den1, Rexp, b1r, g, bt, mu, va, W2, a2s, a2d, n, bn):
    hid = W2.shape[1]
    grid = (n // bn,)
    return pl.pallas_call(
        _mid_body,
        grid=grid,
        in_specs=[
            pl.BlockSpec((2, bn, 128), lambda i: (0, i, 0)),
            pl.BlockSpec((2, bn, 8), lambda i: (0, i, 0)),
            pl.BlockSpec((8, 128), lambda i: (0, 0)),
            pl.BlockSpec((1, 128), lambda i: (0, 0)),
            pl.BlockSpec((1, 128), lambda i: (0, 0)),
            pl.BlockSpec((1, 128), lambda i: (0, 0)),
            pl.BlockSpec((1, 128), lambda i: (0, 0)),
            pl.BlockSpec((1, 128), lambda i: (0, 0)),
            pl.BlockSpec((128, hid), lambda i: (0, 0)),
            pl.BlockSpec((hid, 128), lambda i: (0, 0)),
            pl.BlockSpec((hid, 128), lambda i: (0, 0)),
        ],
        out_specs=[
            pl.BlockSpec((bn, hid), lambda i: (i, 0)),
            pl.BlockSpec((bn, 128), lambda i: (i, 0)),
            pl.BlockSpec((bn, 128), lambda i: (i, 0)),
        ],
        out_shape=[
            jax.ShapeDtypeStruct((n, hid), F32),
            jax.ShapeDtypeStruct((n, 128), F32),
            jax.ShapeDtypeStruct((n, 128), F32),
        ],
    )(acc1, den1, Rexp, b1r, g, bt, mu, va, W2, a2s, a2d)


def _make_edge2(npad, n, e):
    k2 = 512
    npt16 = n * 16 // NS
    nwin_total = e // k2
    nbase = nwin_total // NWORK
    nrem = nwin_total % NWORK
    rows_pt = npad // NS
    mesh = plsc.VectorSubcoreMesh(core_axis_name="c", subcore_axis_name="s")

    @functools.partial(
        pl.kernel,
        mesh=mesh,
        out_type=(
            jax.ShapeDtypeStruct((NC, npad * 16), F32),
            jax.ShapeDtypeStruct((NC, npad), F32),
        ),
        scratch_types=[
            pltpu.VMEM_SHARED((npad * 16,), F32),
            pltpu.VMEM_SHARED((npad,), F32),
            pltpu.VMEM((k2,), F32),
            pltpu.VMEM((k2,), F32),
            pltpu.VMEM((k2,), F32),
            pltpu.VMEM((k2,), jnp.int32),
            pltpu.VMEM((k2,), jnp.int32),
            pltpu.VMEM((k2 * 16,), jnp.int32),
            pltpu.VMEM((k2 * 16,), jnp.int32),
            pltpu.VMEM((k2 * 16,), F32),
            pltpu.VMEM((k2 * 16,), F32),
            pltpu.SemaphoreType.DMA,
        ],
    )
    def edge2(src_hbm, dst_hbm, h2f_hbm, as_hbm, ad_hbm,
              acc_out, den_out,
              acc_sp, den_sp, asr2, adr2, exs, sidx, didx, idxg, idxm,
              h2r, msg, sem):
        cid = lax.axis_index("c")
        sid = lax.axis_index("s")
        wid = sid * NC + cid
        r0 = sid * rows_pt
        lane = lax.iota(jnp.int32, 16)
        fpt = rows_pt * 16
        chunk = 2048
        nch = fpt // chunk

        def zb(i, c):
            msg[pl.ds(i * 16, 16)] = jnp.zeros((16,), F32)
            return c

        lax.fori_loop(0, chunk // 16, zb, 0)

        def zex(i, c):
            exs[pl.ds(i * 16, 16)] = jnp.zeros((16,), F32)
            return c

        lax.fori_loop(0, k2 // 16, zex, 0)
        for c in range(nch):
            pltpu.sync_copy(msg.at[pl.ds(0, chunk)],
                            acc_sp.at[pl.ds(r0 * 16 + c * chunk, chunk)])
        for c in range(rows_pt // k2 if rows_pt >= k2 else 0):
            pltpu.sync_copy(exs, den_sp.at[pl.ds(r0 + c * k2, k2)])
        for c in range(rows_pt % k2 // 128):
            pltpu.sync_copy(
                exs.at[pl.ds(0, 128)],
                den_sp.at[pl.ds(r0 + (rows_pt // k2) * k2 + c * 128, 128)])
        plsc.subcore_barrier()

        nwin = nbase + jnp.where(wid < nrem, 1, 0)

        def window(w, carry):
            off = (wid + NWORK * w) * k2
            c1 = pltpu.async_copy(src_hbm.at[pl.ds(off, k2)], sidx, sem)
            c2 = pltpu.async_copy(dst_hbm.at[pl.ds(off, k2)], didx, sem)
            c1.wait()
            c2.wait()
            ga = pltpu.async_copy(as_hbm.at[sidx], asr2, sem)
            gb = pltpu.async_copy(ad_hbm.at[didx], adr2, sem)

            def idx_group(j, c):
                svec = sidx[pl.ds(j * 16, 16)]
                dvec = didx[pl.ds(j * 16, 16)]
                for t in range(16):
                    k = j * 16 + t
                    idxg[pl.ds(k * 16, 16)] = svec[t] * 16 + lane
                    idxm[pl.ds(k * 16, 16)] = dvec[t] * 16 + lane
                return c

            lax.fori_loop(0, k2 // 16, idx_group, 0)
            gh = pltpu.async_copy(h2f_hbm.at[idxg], h2r, sem)
            ga.wait()
            gb.wait()
            gh.wait()

            def per_group(j, c):
                sv = asr2[pl.ds(j * 16, 16)]
                dv = adr2[pl.ds(j * 16, 16)]
                v = sv + dv
                v = jnp.where(v > 0, v, v * 0.2)
                ex = jnp.exp(v)
                exs[pl.ds(j * 16, 16)] = ex
                for t in range(16):
                    k = j * 16 + t
                    msg[pl.ds(k * 16, 16)] = h2r[pl.ds(k * 16, 16)] * ex[t]
                return c

            lax.fori_loop(0, k2 // 16, per_group, 0)
            s1 = pltpu.async_copy(msg, acc_sp.at[idxm], sem, add=True)
            s2 = pltpu.async_copy(exs, den_sp.at[didx], sem, add=True)
            s1.wait()
            s2.wait()
            return carry

        lax.fori_loop(0, nwin, window, 0)
        plsc.subcore_barrier()
        for c in range(nch):
            f0 = r0 * 16 + c * chunk
            pltpu.sync_copy(acc_sp.at[pl.ds(f0, chunk)],
                            msg.at[pl.ds(0, chunk)])
            pltpu.sync_copy(msg.at[pl.ds(0, chunk)],
                            acc_out.at[cid, pl.ds(f0, chunk)])
        for c in range(rows_pt // 128):
            f0 = r0 + c * 128
            pltpu.sync_copy(den_sp.at[pl.ds(f0, 128)], exs.at[pl.ds(0, 128)])
            pltpu.sync_copy(exs.at[pl.ds(0, 128)],
                            den_out.at[cid, pl.ds(f0, 128)])

    return edge2


def _head_body(acc_ref, den_ref, ones_ref, b2_ref, wc_ref, bc_ref, ws_ref,
               bs_ref, wh_ref, bh_ref, we_ref, be_ref,
               main_ref, sim_ref, hom_ref, ent_ref):
    num = acc_ref[0] + acc_ref[1]
    d = den_ref[0] + den_ref[1]
    den = jnp.dot(d, ones_ref[...], preferred_element_type=F32)
    h = num / (den + 1e-16) + b2_ref[...]
    mo = jnp.dot(h, wc_ref[...], preferred_element_type=F32) + bc_ref[...]
    mo = mo - jnp.max(mo, axis=1, keepdims=True)
    main_ref[...] = mo - jnp.log(jnp.sum(jnp.exp(mo), axis=1, keepdims=True))
    so = jnp.dot(h, ws_ref[...], preferred_element_type=F32) + bs_ref[...]
    so = jnp.exp(so - jnp.max(so, axis=1, keepdims=True))
    sim_ref[...] = so / jnp.sum(so, axis=1, keepdims=True)
    hom_ref[...] = jax.nn.sigmoid(
        jnp.dot(h, wh_ref[...], preferred_element_type=F32) + bh_ref[...])
    ent_ref[...] = jax.nn.sigmoid(
        jnp.dot(h, we_ref[...], preferred_element_type=F32) + be_ref[...])


def _heads(acc2, den2, ones16, b2r, Wc, bcr, Ws, bsr, Wh, bhr, We, ber,
           n, bn):
    out_c = Wc.shape[1]
    ncls = Ws.shape[1]
    grid = (n // bn,)

    def cst(shp):
        return pl.BlockSpec(shp, lambda *_: tuple(0 for _ in shp))

    return pl.pallas_call(
        _head_body,
        grid=grid,
        in_specs=[
            pl.BlockSpec((2, bn, 16), lambda i: (0, i, 0)),
            pl.BlockSpec((2, bn, 1), lambda i: (0, i, 0)),
            cst((1, 16)), cst((1, 16)),
            cst((16, out_c)), cst((1, out_c)),
            cst((16, ncls)), cst((1, ncls)),
            cst((16, 1)), cst((1, 1)),
            cst((16, 1)), cst((1, 1)),
        ],
        out_specs=[
            pl.BlockSpec((bn, out_c), lambda i: (i, 0)),
            pl.BlockSpec((bn, ncls), lambda i: (i, 0)),
            pl.BlockSpec((bn, 1), lambda i: (i, 0)),
            pl.BlockSpec((bn, 1), lambda i: (i, 0)),
        ],
        out_shape=[
            jax.ShapeDtypeStruct((n, out_c), F32),
            jax.ShapeDtypeStruct((n, ncls), F32),
            jax.ShapeDtypeStruct((n, 1), F32),
            jax.ShapeDtypeStruct((n, 1), F32),
        ],
    )(acc2, den2, ones16, b2r, Wc, bcr, Ws, bsr, Wh, bhr, We, ber)


def kernel(x, edge_index, W1, att_src1, att_dst1, b1, bn_gamma, bn_beta,
           bn_mean, bn_var, W2, att_src2, att_dst2, b2, Wc, bc, Ws, bs,
           Wh, bh, We, be):
    n, din = x.shape
    e = edge_index.shape[1]
    heads, hid = att_src1.shape
    dh = heads * hid

    src = edge_index[0]
    dst = edge_index[1]

    A1s = jnp.zeros((dh, 16), F32)
    A1d = jnp.zeros((dh, 16), F32)
    for h in range(heads):
        A1s = A1s.at[h * hid:(h + 1) * hid, h].set(att_src1[h])
        A1d = A1d.at[h * hid:(h + 1) * hid, h].set(att_dst1[h])

    rexp = np.zeros((8, dh), np.float32)
    for h in range(heads):
        rexp[h, h * hid:(h + 1) * hid] = 1.0
    Rexp = jnp.asarray(rexp)

    bn = 1000
    npad = 10240 if n == 10000 else ((n + NS * 128 - 1) // (NS * 128)) * NS * 128
    h1, as1, ad1 = _dense1(x, W1, A1s, A1d, bn)

    edge1 = _make_edge1(npad, n, e)
    alcat = jnp.concatenate([as1.reshape(-1), ad1.reshape(-1)])
    acc1, den1f = edge1(src, dst, h1, alcat)
    den1 = den1f.reshape(NC, npad, 8)

    h2, as2, ad2 = _mid(
        acc1, den1, Rexp, b1.reshape(1, dh), bn_gamma.reshape(1, dh),
        bn_beta.reshape(1, dh), bn_mean.reshape(1, dh), bn_var.reshape(1, dh),
        W2, jnp.tile(att_src2.reshape(hid, 1), (1, 128)),
        jnp.tile(att_dst2.reshape(hid, 1), (1, 128)), n, bn)

    edge2 = _make_edge2(npad, n, e)
    acc2f, den2f = edge2(src, dst, h2.reshape(-1), as2[:, 0], ad2[:, 0])
    acc2 = acc2f.reshape(NC, npad, 16)
    den2 = den2f.reshape(NC, npad, 1)

    main, sim, hom, ent = _heads(
        acc2, den2, jnp.ones((1, 16), F32), b2.reshape(1, hid),
        Wc, bc.reshape(1, -1),
        Ws, bs.reshape(1, -1), Wh, bh.reshape(1, 1), We, be.reshape(1, 1),
        n, bn)
    return main, sim, hom[:, 0], ent[:, 0]


# speedup vs baseline: 52.7153x; 1.0285x over previous
import functools
import numpy as np
import jax
import jax.numpy as jnp
from jax import lax
from jax.experimental import pallas as pl
from jax.experimental.pallas import tpu as pltpu
from jax.experimental.pallas import tpu_sc as plsc

NC = 2
NS = 16
NWORK = NC * NS

F32 = jnp.float32

_GDN = jax.lax.GatherDimensionNumbers(
    offset_dims=(), collapsed_slice_dims=(0,), start_index_map=(0,))


def _vtake(vec, idx):
    return lax.gather(vec, idx[:, None], _GDN, (1,),
                      mode=lax.GatherScatterMode.PROMISE_IN_BOUNDS)


def _dense1_body(x_ref, w1_ref, a1s_ref, a1d_ref, h_ref, as_ref, ad_ref):
    h = jnp.dot(x_ref[...], w1_ref[...], preferred_element_type=F32)
    h_ref[...] = h
    as_ref[...] = jnp.dot(h, a1s_ref[...], preferred_element_type=F32)
    ad_ref[...] = jnp.dot(h, a1d_ref[...], preferred_element_type=F32)


def _dense1(x, W1, A1s, A1d, bn):
    n, din = x.shape
    dh = W1.shape[1]
    grid = (n // bn,)
    return pl.pallas_call(
        _dense1_body,
        grid=grid,
        in_specs=[
            pl.BlockSpec((bn, din), lambda i: (i, 0)),
            pl.BlockSpec((din, dh), lambda i: (0, 0)),
            pl.BlockSpec((dh, 16), lambda i: (0, 0)),
            pl.BlockSpec((dh, 16), lambda i: (0, 0)),
        ],
        out_specs=[
            pl.BlockSpec((bn, dh), lambda i: (i, 0)),
            pl.BlockSpec((bn, 16), lambda i: (i, 0)),
            pl.BlockSpec((bn, 16), lambda i: (i, 0)),
        ],
        out_shape=[
            jax.ShapeDtypeStruct((n, dh), F32),
            jax.ShapeDtypeStruct((n, 16), F32),
            jax.ShapeDtypeStruct((n, 16), F32),
        ],
    )(x, W1, A1s, A1d)


def _make_edge1(npad, n, e):
    k1 = 160
    n16 = n * 16
    nwin_total = e // k1
    nbase = nwin_total // NWORK
    nrem = nwin_total % NWORK
    rows_pt = npad // NS
    mesh = plsc.VectorSubcoreMesh(core_axis_name="c", subcore_axis_name="s")

    @functools.partial(
        pl.kernel,
        mesh=mesh,
        out_type=(
            jax.ShapeDtypeStruct((NC, npad, 128), F32),
            jax.ShapeDtypeStruct((NC, npad * 8), F32),
        ),
        scratch_types=[
            pltpu.VMEM_SHARED((npad, 128), F32),
            pltpu.VMEM_SHARED((npad * 8,), F32),
            pltpu.VMEM((k1,), jnp.int32),
            pltpu.VMEM((k1 // 2,), jnp.int32),
            pltpu.VMEM((k1 // 2,), jnp.int32),
            pltpu.VMEM((k1 * 8,), jnp.int32),
            pltpu.VMEM((k1 * 8,), jnp.int32),
            pltpu.VMEM((k1 * 4,), jnp.int32),
            pltpu.VMEM((k1 * 4,), jnp.int32),
            pltpu.VMEM((k1 * 8,), F32),
            pltpu.VMEM((k1 * 8,), F32),
            pltpu.VMEM((k1 * 4,), F32),
            pltpu.VMEM((k1 * 4,), F32),
            pltpu.VMEM((k1 // 2, 128), F32),
            pltpu.VMEM((k1 // 2, 128), F32),
            pltpu.SemaphoreType.DMA,
            pltpu.SemaphoreType.DMA,
        ],
    )
    def edge1(src_hbm, dst_hbm, h_hbm, alcat_hbm,
              acc_out, den_out,
              acc_sp, den_sp, sidx, didxA, didxB, idx1A, idx1B, idx2A,
              idx2B, asrA, asrB, exrA, exrB, hrowsA, hrowsB, semg, sems):
        cid = lax.axis_index("c")
        sid = lax.axis_index("s")
        wid = sid * NC + cid
        r0 = sid * rows_pt
        lane = lax.iota(jnp.int32, 16)

        kh = k1 // 2
        mlo = lane < 8
        la7 = lane & 7

        def zrow(r, c):
            for hh in range(8):
                hrowsA[r, pl.ds(hh * 16, 16)] = jnp.zeros((16,), F32)
            return c

        lax.fori_loop(0, kh, zrow, 0)

        def zex(i, c):
            exrA[pl.ds(i * 16, 16)] = jnp.zeros((16,), F32)
            return c

        lax.fori_loop(0, k1 * 4 // 16, zex, 0)
        for c in range(rows_pt // kh):
            r = r0 + c * kh
            pltpu.sync_copy(hrowsA, acc_sp.at[pl.ds(r, kh)])
        for c in range(rows_pt * 8 // (k1 * 4)):
            f0 = r0 * 8 + c * k1 * 4
            pltpu.sync_copy(exrA, den_sp.at[pl.ds(f0, k1 * 4)])
        plsc.subcore_barrier()

        nwin = nbase + jnp.where(wid < nrem, 1, 0)

        def build_idx(h0, didxX, idx1X, idx2X):
            def idx_group(j, c):
                svec = sidx[pl.ds(h0 + j * 16, 16)]
                dvec = didxX[pl.ds(j * 16, 16)]
                for t in range(16):
                    k = j * 16 + t
                    idx1X[pl.ds(k * 16, 16)] = (
                        jnp.where(mlo, svec[t] * 16, n16 + dvec[t] * 16)
                        + la7)
                    if t % 2 == 0:
                        p = j * 8 + t // 2
                        idx2X[pl.ds(p * 16, 16)] = (
                            jnp.where(mlo, dvec[t] * 8, dvec[t + 1] * 8)
                            + la7)
                return c

            lax.fori_loop(0, kh // 16, idx_group, 0)

        def compute(asrX, exrX, hrowsX):
            def per_pair(p, c):
                e0 = 2 * p
                e1 = 2 * p + 1
                g0 = asrX[pl.ds(e0 * 16, 16)]
                v0 = _vtake(g0, la7) + _vtake(g0, la7 + 8)
                v0 = jnp.where(v0 > 0, v0, v0 * 0.2)
                ex0 = jnp.exp(v0)
                g1 = asrX[pl.ds(e1 * 16, 16)]
                v1 = _vtake(g1, la7) + _vtake(g1, la7 + 8)
                v1 = jnp.where(v1 > 0, v1, v1 * 0.2)
                ex1 = jnp.exp(v1)
                exrX[pl.ds(p * 16, 16)] = jnp.where(mlo, ex0, ex1)
                for hh in range(8):
                    hrowsX[e0, pl.ds(hh * 16, 16)] = (
                        hrowsX[e0, pl.ds(hh * 16, 16)] * ex0[hh])
                    hrowsX[e1, pl.ds(hh * 16, 16)] = (
                        hrowsX[e1, pl.ds(hh * 16, 16)] * ex1[hh])
                return c

            lax.fori_loop(0, kh // 2, per_pair, 0)

        def window(w, carry):
            off = (wid + NWORK * w) * k1
            c1 = pltpu.async_copy(src_hbm.at[pl.ds(off, k1)], sidx, semg)
            c2 = pltpu.async_copy(dst_hbm.at[pl.ds(off, kh)], didxA, semg)
            c3 = pltpu.async_copy(dst_hbm.at[pl.ds(off + kh, kh)], didxB,
                                  semg)
            c1.wait()
            c2.wait()
            c3.wait()
            ghA = pltpu.async_copy(h_hbm.at[sidx.at[pl.ds(0, kh)]], hrowsA,
                                   semg)
            ghB = pltpu.async_copy(h_hbm.at[sidx.at[pl.ds(kh, kh)]], hrowsB,
                                   semg)
            build_idx(0, didxA, idx1A, idx2A)
            gaA = pltpu.async_copy(alcat_hbm.at[idx1A], asrA, semg)
            build_idx(kh, didxB, idx1B, idx2B)
            gaB = pltpu.async_copy(alcat_hbm.at[idx1B], asrB, semg)
            gaA.wait()
            ghA.wait()
            compute(asrA, exrA, hrowsA)
            sA1 = pltpu.async_copy(exrA, den_sp.at[idx2A], sems, add=True)
            sA2 = pltpu.async_copy(hrowsA, acc_sp.at[didxA], sems, add=True)
            gaB.wait()
            ghB.wait()
            compute(asrB, exrB, hrowsB)
            sB1 = pltpu.async_copy(exrB, den_sp.at[idx2B], sems, add=True)
            sB2 = pltpu.async_copy(hrowsB, acc_sp.at[didxB], sems, add=True)
            sA1.wait()
            sA2.wait()
            sB1.wait()
            sB2.wait()
            return carry

        lax.fori_loop(0, nwin, window, 0)
        plsc.subcore_barrier()

        for c in range(rows_pt // kh):
            r = r0 + c * kh
            pltpu.sync_copy(acc_sp.at[pl.ds(r, kh)], hrowsA)
            pltpu.sync_copy(hrowsA, acc_out.at[cid, pl.ds(r, kh)])
        for c in range(rows_pt * 8 // (k1 * 4)):
            f0 = r0 * 8 + c * k1 * 4
            pltpu.sync_copy(den_sp.at[pl.ds(f0, k1 * 4)], exrA)
            pltpu.sync_copy(exrA, den_out.at[cid, pl.ds(f0, k1 * 4)])

    return edge1


def _mid_body(acc_ref, den_ref, rexp_ref, b1_ref, g_ref, bt_ref, mu_ref,
              va_ref, w2_ref, a2s_ref, a2d_ref, h2_ref, as2_ref, ad2_ref):
    a = acc_ref[0] + acc_ref[1]
    d = den_ref[0] + den_ref[1]
    dexp = jnp.dot(d, rexp_ref[...], preferred_element_type=F32)
    h1 = a / (dexp + 1e-16) + b1_ref[...]
    scale = g_ref[...] * lax.rsqrt(va_ref[...] + 1e-5)
    h1 = (h1 - mu_ref[...]) * scale + bt_ref[...]
    h1 = jnp.where(h1 > 0, h1, jnp.exp(h1) - 1.0)
    h2 = jnp.dot(h1, w2_ref[...], preferred_element_type=F32)
    h2_ref[...] = h2
    as2_ref[...] = jnp.dot(h2, a2s_ref[...], preferred_element_type=F32)
    ad2_ref[...] = jnp.dot(h2, a2d_ref[...], preferred_element_type=F32)


def _mid(acc1, den1, Rexp, b1r, g, bt, mu, va, W2, a2s, a2d, n, bn):
    hid = W2.shape[1]
    grid = (n // bn,)
    return pl.pallas_call(
        _mid_body,
        grid=grid,
        in_specs=[
            pl.BlockSpec((2, bn, 128), lambda i: (0, i, 0)),
            pl.BlockSpec((2, bn, 8), lambda i: (0, i, 0)),
            pl.BlockSpec((8, 128), lambda i: (0, 0)),
            pl.BlockSpec((1, 128), lambda i: (0, 0)),
            pl.BlockSpec((1, 128), lambda i: (0, 0)),
            pl.BlockSpec((1, 128), lambda i: (0, 0)),
            pl.BlockSpec((1, 128), lambda i: (0, 0)),
            pl.BlockSpec((1, 128), lambda i: (0, 0)),
            pl.BlockSpec((128, hid), lambda i: (0, 0)),
            pl.BlockSpec((hid, 128), lambda i: (0, 0)),
            pl.BlockSpec((hid, 128), lambda i: (0, 0)),
        ],
        out_specs=[
            pl.BlockSpec((bn, hid), lambda i: (i, 0)),
            pl.BlockSpec((bn, 128), lambda i: (i, 0)),
            pl.BlockSpec((bn, 128), lambda i: (i, 0)),
        ],
        out_shape=[
            jax.ShapeDtypeStruct((n, hid), F32),
            jax.ShapeDtypeStruct((n, 128), F32),
            jax.ShapeDtypeStruct((n, 128), F32),
        ],
    )(acc1, den1, Rexp, b1r, g, bt, mu, va, W2, a2s, a2d)


def _make_edge2(npad, n, e):
    k2 = 512
    npt16 = n * 16 // NS
    nwin_total = e // k2
    nbase = nwin_total // NWORK
    nrem = nwin_total % NWORK
    rows_pt = npad // NS
    mesh = plsc.VectorSubcoreMesh(core_axis_name="c", subcore_axis_name="s")

    @functools.partial(
        pl.kernel,
        mesh=mesh,
        out_type=(
            jax.ShapeDtypeStruct((NC, npad * 16), F32),
            jax.ShapeDtypeStruct((NC, npad), F32),
        ),
        scratch_types=[
            pltpu.VMEM_SHARED((npad * 16,), F32),
            pltpu.VMEM_SHARED((npad,), F32),
            pltpu.VMEM((k2 // 2,), F32),
            pltpu.VMEM((k2 // 2,), F32),
            pltpu.VMEM((k2 // 2,), F32),
            pltpu.VMEM((k2 // 2,), F32),
            pltpu.VMEM((k2 // 2,), F32),
            pltpu.VMEM((k2 // 2,), F32),
            pltpu.VMEM((k2,), jnp.int32),
            pltpu.VMEM((k2 // 2,), jnp.int32),
            pltpu.VMEM((k2 // 2,), jnp.int32),
            pltpu.VMEM((k2 * 8,), jnp.int32),
            pltpu.VMEM((k2 * 8,), jnp.int32),
            pltpu.VMEM((k2 * 8,), jnp.int32),
            pltpu.VMEM((k2 * 8,), jnp.int32),
            pltpu.VMEM((k2 * 8,), F32),
            pltpu.VMEM((k2 * 8,), F32),
            pltpu.VMEM((k2 * 8,), F32),
            pltpu.VMEM((k2 * 8,), F32),
            pltpu.SemaphoreType.DMA,
            pltpu.SemaphoreType.DMA,
        ],
    )
    def edge2(src_hbm, dst_hbm, h2f_hbm, as_hbm, ad_hbm,
              acc_out, den_out,
              acc_sp, den_sp, asrA, asrB, adrA, adrB, exsA, exsB,
              sidx, didxA, didxB, idxgA, idxgB, idxmA, idxmB,
              h2rA, h2rB, msgA, msgB, semg, sems):
        cid = lax.axis_index("c")
        sid = lax.axis_index("s")
        wid = sid * NC + cid
        r0 = sid * rows_pt
        lane = lax.iota(jnp.int32, 16)
        kh = k2 // 2
        fpt = rows_pt * 16
        chunk = 2048
        nch = fpt // chunk

        def zb(i, c):
            msgA[pl.ds(i * 16, 16)] = jnp.zeros((16,), F32)
            return c

        lax.fori_loop(0, chunk // 16, zb, 0)

        def zex(i, c):
            exsA[pl.ds(i * 16, 16)] = jnp.zeros((16,), F32)
            return c

        lax.fori_loop(0, kh // 16, zex, 0)
        for c in range(nch):
            pltpu.sync_copy(msgA.at[pl.ds(0, chunk)],
                            acc_sp.at[pl.ds(r0 * 16 + c * chunk, chunk)])
        for c in range(rows_pt // 128):
            pltpu.sync_copy(exsA.at[pl.ds(0, 128)],
                            den_sp.at[pl.ds(r0 + c * 128, 128)])
        plsc.subcore_barrier()

        nwin = nbase + jnp.where(wid < nrem, 1, 0)

        def build_idx(h0, didxX, idxgX, idxmX):
            def idx_group(j, c):
                svec = sidx[pl.ds(h0 + j * 16, 16)]
                dvec = didxX[pl.ds(j * 16, 16)]
                for t in range(16):
                    k = j * 16 + t
                    idxgX[pl.ds(k * 16, 16)] = svec[t] * 16 + lane
                    idxmX[pl.ds(k * 16, 16)] = dvec[t] * 16 + lane
                return c

            lax.fori_loop(0, kh // 16, idx_group, 0)

        def compute(asrX, adrX, exsX, h2rX, msgX):
            def per_group(j, c):
                sv = asrX[pl.ds(j * 16, 16)]
                dv = adrX[pl.ds(j * 16, 16)]
                v = sv + dv
                v = jnp.where(v > 0, v, v * 0.2)
                ex = jnp.exp(v)
                exsX[pl.ds(j * 16, 16)] = ex
                for t in range(16):
                    k = j * 16 + t
                    msgX[pl.ds(k * 16, 16)] = h2rX[pl.ds(k * 16, 16)] * ex[t]
                return c

            lax.fori_loop(0, kh // 16, per_group, 0)

        def window(w, carry):
            off = (wid + NWORK * w) * k2
            c1 = pltpu.async_copy(src_hbm.at[pl.ds(off, k2)], sidx, semg)
            c2 = pltpu.async_copy(dst_hbm.at[pl.ds(off, kh)], didxA, semg)
            c3 = pltpu.async_copy(dst_hbm.at[pl.ds(off + kh, kh)], didxB,
                                  semg)
            c1.wait()
            c2.wait()
            c3.wait()
            gaA = pltpu.async_copy(as_hbm.at[sidx.at[pl.ds(0, kh)]], asrA,
                                   semg)
            gbA = pltpu.async_copy(ad_hbm.at[didxA], adrA, semg)
            gaB = pltpu.async_copy(as_hbm.at[sidx.at[pl.ds(kh, kh)]], asrB,
                                   semg)
            gbB = pltpu.async_copy(ad_hbm.at[didxB], adrB, semg)
            build_idx(0, didxA, idxgA, idxmA)
            ghA = pltpu.async_copy(h2f_hbm.at[idxgA], h2rA, semg)
            build_idx(kh, didxB, idxgB, idxmB)
            ghB = pltpu.async_copy(h2f_hbm.at[idxgB], h2rB, semg)
            gaA.wait()
            gbA.wait()
            ghA.wait()
            compute(asrA, adrA, exsA, h2rA, msgA)
            sA1 = pltpu.async_copy(msgA, acc_sp.at[idxmA], sems, add=True)
            sA2 = pltpu.async_copy(exsA, den_sp.at[didxA], sems, add=True)
            gaB.wait()
            gbB.wait()
            ghB.wait()
            compute(asrB, adrB, exsB, h2rB, msgB)
            sB1 = pltpu.async_copy(msgB, acc_sp.at[idxmB], sems, add=True)
            sB2 = pltpu.async_copy(exsB, den_sp.at[didxB], sems, add=True)
            sA1.wait()
            sA2.wait()
            sB1.wait()
            sB2.wait()
            return carry

        lax.fori_loop(0, nwin, window, 0)
        plsc.subcore_barrier()
        for c in range(nch):
            f0 = r0 * 16 + c * chunk
            pltpu.sync_copy(acc_sp.at[pl.ds(f0, chunk)],
                            msgA.at[pl.ds(0, chunk)])
            pltpu.sync_copy(msgA.at[pl.ds(0, chunk)],
                            acc_out.at[cid, pl.ds(f0, chunk)])
        for c in range(rows_pt // 128):
            f0 = r0 + c * 128
            pltpu.sync_copy(den_sp.at[pl.ds(f0, 128)], exsA.at[pl.ds(0, 128)])
            pltpu.sync_copy(exsA.at[pl.ds(0, 128)],
                            den_out.at[cid, pl.ds(f0, 128)])

    return edge2


def _head_body(acc_ref, den_ref, ones_ref, b2_ref, wc_ref, bc_ref, ws_ref,
               bs_ref, wh_ref, bh_ref, we_ref, be_ref,
               main_ref, sim_ref, hom_ref, ent_ref):
    num = acc_ref[0] + acc_ref[1]
    d = den_ref[0] + den_ref[1]
    den = jnp.dot(d, ones_ref[...], preferred_element_type=F32)
    h = num / (den + 1e-16) + b2_ref[...]
    mo = jnp.dot(h, wc_ref[...], preferred_element_type=F32) + bc_ref[...]
    mo = mo - jnp.max(mo, axis=1, keepdims=True)
    main_ref[...] = mo - jnp.log(jnp.sum(jnp.exp(mo), axis=1, keepdims=True))
    so = jnp.dot(h, ws_ref[...], preferred_element_type=F32) + bs_ref[...]
    so = jnp.exp(so - jnp.max(so, axis=1, keepdims=True))
    sim_ref[...] = so / jnp.sum(so, axis=1, keepdims=True)
    hom_ref[...] = jax.nn.sigmoid(
        jnp.dot(h, wh_ref[...], preferred_element_type=F32) + bh_ref[...])
    ent_ref[...] = jax.nn.sigmoid(
        jnp.dot(h, we_ref[...], preferred_element_type=F32) + be_ref[...])


def _heads(acc2, den2, ones16, b2r, Wc, bcr, Ws, bsr, Wh, bhr, We, ber,
           n, bn):
    out_c = Wc.shape[1]
    ncls = Ws.shape[1]
    grid = (n // bn,)

    def cst(shp):
        return pl.BlockSpec(shp, lambda *_: tuple(0 for _ in shp))

    return pl.pallas_call(
        _head_body,
        grid=grid,
        in_specs=[
            pl.BlockSpec((2, bn, 16), lambda i: (0, i, 0)),
            pl.BlockSpec((2, bn, 1), lambda i: (0, i, 0)),
            cst((1, 16)), cst((1, 16)),
            cst((16, out_c)), cst((1, out_c)),
            cst((16, ncls)), cst((1, ncls)),
            cst((16, 1)), cst((1, 1)),
            cst((16, 1)), cst((1, 1)),
        ],
        out_specs=[
            pl.BlockSpec((bn, out_c), lambda i: (i, 0)),
            pl.BlockSpec((bn, ncls), lambda i: (i, 0)),
            pl.BlockSpec((bn, 1), lambda i: (i, 0)),
            pl.BlockSpec((bn, 1), lambda i: (i, 0)),
        ],
        out_shape=[
            jax.ShapeDtypeStruct((n, out_c), F32),
            jax.ShapeDtypeStruct((n, ncls), F32),
            jax.ShapeDtypeStruct((n, 1), F32),
            jax.ShapeDtypeStruct((n, 1), F32),
        ],
    )(acc2, den2, ones16, b2r, Wc, bcr, Ws, bsr, Wh, bhr, We, ber)


def kernel(x, edge_index, W1, att_src1, att_dst1, b1, bn_gamma, bn_beta,
           bn_mean, bn_var, W2, att_src2, att_dst2, b2, Wc, bc, Ws, bs,
           Wh, bh, We, be):
    n, din = x.shape
    e = edge_index.shape[1]
    heads, hid = att_src1.shape
    dh = heads * hid

    src = edge_index[0]
    dst = edge_index[1]

    A1s = jnp.zeros((dh, 16), F32)
    A1d = jnp.zeros((dh, 16), F32)
    for h in range(heads):
        A1s = A1s.at[h * hid:(h + 1) * hid, h].set(att_src1[h])
        A1d = A1d.at[h * hid:(h + 1) * hid, h].set(att_dst1[h])

    rexp = np.zeros((8, dh), np.float32)
    for h in range(heads):
        rexp[h, h * hid:(h + 1) * hid] = 1.0
    Rexp = jnp.asarray(rexp)

    bn = 1000
    npad = 10240 if n == 10000 else ((n + NS * 128 - 1) // (NS * 128)) * NS * 128
    h1, as1, ad1 = _dense1(x, W1, A1s, A1d, bn)

    edge1 = _make_edge1(npad, n, e)
    alcat = jnp.concatenate([as1.reshape(-1), ad1.reshape(-1)])
    acc1, den1f = edge1(src, dst, h1, alcat)
    den1 = den1f.reshape(NC, npad, 8)

    h2, as2, ad2 = _mid(
        acc1, den1, Rexp, b1.reshape(1, dh), bn_gamma.reshape(1, dh),
        bn_beta.reshape(1, dh), bn_mean.reshape(1, dh), bn_var.reshape(1, dh),
        W2, jnp.tile(att_src2.reshape(hid, 1), (1, 128)),
        jnp.tile(att_dst2.reshape(hid, 1), (1, 128)), n, bn)

    edge2 = _make_edge2(npad, n, e)
    acc2f, den2f = edge2(src, dst, h2.reshape(-1), as2[:, 0], ad2[:, 0])
    acc2 = acc2f.reshape(NC, npad, 16)
    den2 = den2f.reshape(NC, npad, 1)

    main, sim, hom, ent = _heads(
        acc2, den2, jnp.ones((1, 16), F32), b2.reshape(1, hid),
        Wc, bc.reshape(1, -1),
        Ws, bs.reshape(1, -1), Wh, bh.reshape(1, 1), We, be.reshape(1, 1),
        n, bn)
    return main, sim, hom[:, 0], ent[:, 0]
